# Initial kernel scaffold; baseline (speedup 1.0000x reference)
import functools

import jax
import jax.numpy as jnp
from jax import lax
from jax.experimental import pallas as pl
from jax.experimental.pallas import tpu as pltpu
from jax.experimental.pallas import tpu_sc as plsc

N = 10000
E = 320000
D = 128

NC = 2
NS = 16
NW = NC * NS
EPW = E // NW
CHUNK = 80
NCHUNK = EPW // CHUNK
ROWS_PER_SUB = N // NS


def _make_sc_spmm(width: int):
  mesh = plsc.VectorSubcoreMesh(core_axis_name="c", subcore_axis_name="s")

  @functools.partial(
      pl.kernel,
      out_type=jax.ShapeDtypeStruct((NC, N, width), jnp.float32),
      mesh=mesh,
      scratch_types=[
          pltpu.VMEM((CHUNK,), jnp.int32),
          pltpu.VMEM((CHUNK,), jnp.int32),
          pltpu.VMEM((CHUNK, width), jnp.float32),
          pltpu.VMEM_SHARED((N, width), jnp.float32),
          pltpu.SemaphoreType.DMA,
      ],
  )
  def spmm(table_hbm, edge_hbm, zeros_hbm, out_hbm, idx_src, idx_dst, rows,
           acc, sem):
    c = lax.axis_index("c")
    s = lax.axis_index("s")
    wid = s * NC + c
    stripe = pl.ds(s * ROWS_PER_SUB, ROWS_PER_SUB)
    pltpu.sync_copy(zeros_hbm.at[stripe], acc.at[stripe])
    plsc.subcore_barrier()

    base = wid * EPW

    def body(j, carry):
      off = base + j * CHUNK
      pltpu.sync_copy(edge_hbm.at[0, pl.ds(off, CHUNK)], idx_src)
      pltpu.sync_copy(edge_hbm.at[1, pl.ds(off, CHUNK)], idx_dst)
      pltpu.async_copy(table_hbm.at[idx_src], rows, sem).wait()
      pltpu.sync_copy(rows, acc.at[idx_dst], add=True)
      return carry

    lax.fori_loop(0, NCHUNK, body, 0)
    plsc.subcore_barrier()
    pltpu.sync_copy(acc.at[stripe], out_hbm.at[c, stripe])

  return spmm


_sc_spmm32 = _make_sc_spmm(32)
_sc_spmm64 = _make_sc_spmm(64)
_sc_spmm16 = _make_sc_spmm(16)


_BN = 1000


def _mm1_body(x_ref, w_ref, o_ref):
  o_ref[...] = jnp.dot(x_ref[...], w_ref[...],
                       preferred_element_type=jnp.float32)


def _tc_mm1(x, w1):
  return pl.pallas_call(
      _mm1_body,
      grid=(N // _BN,),
      in_specs=[
          pl.BlockSpec((_BN, D), lambda i: (i, 0)),
          pl.BlockSpec((D, 32), lambda i: (0, 0)),
      ],
      out_specs=pl.BlockSpec((_BN, 32), lambda i: (i, 0)),
      out_shape=jax.ShapeDtypeStruct((N, 32), jnp.float32),
  )(x, w1)


def _stage_b_body(p0_ref, p1_ref, w_ref, o_ref):
  h = jnp.maximum(p0_ref[...] + p1_ref[...], 0.0)
  o_ref[...] = jnp.dot(h, w_ref[...], preferred_element_type=jnp.float32)


def _tc_stage_b(p0, p1, w23):
  return pl.pallas_call(
      _stage_b_body,
      grid=(N // _BN,),
      in_specs=[
          pl.BlockSpec((_BN, 32), lambda i: (i, 0)),
          pl.BlockSpec((_BN, 32), lambda i: (i, 0)),
          pl.BlockSpec((32, 64), lambda i: (0, 0)),
      ],
      out_specs=pl.BlockSpec((_BN, 64), lambda i: (i, 0)),
      out_shape=jax.ShapeDtypeStruct((N, 64), jnp.float32),
  )(p0, p1, w23)


def _stage_c_body(q0_ref, q1_ref, wd_ref, mu_ref, lv_ref, d_ref):
  sm = q0_ref[...] + q1_ref[...]
  mu = sm[:, :32]
  mu_ref[...] = mu
  lv_ref[...] = sm[:, 32:]
  d_ref[...] = jnp.dot(mu, wd_ref[...], preferred_element_type=jnp.float32)


def _tc_stage_c(q0, q1, wd16):
  return pl.pallas_call(
      _stage_c_body,
      grid=(N // _BN,),
      in_specs=[
          pl.BlockSpec((_BN, 64), lambda i: (i, 0)),
          pl.BlockSpec((_BN, 64), lambda i: (i, 0)),
          pl.BlockSpec((32, 16), lambda i: (0, 0)),
      ],
      out_specs=[
          pl.BlockSpec((_BN, 32), lambda i: (i, 0)),
          pl.BlockSpec((_BN, 32), lambda i: (i, 0)),
          pl.BlockSpec((_BN, 16), lambda i: (i, 0)),
      ],
      out_shape=[
          jax.ShapeDtypeStruct((N, 32), jnp.float32),
          jax.ShapeDtypeStruct((N, 32), jnp.float32),
          jax.ShapeDtypeStruct((N, 16), jnp.float32),
      ],
  )(q0, q1, wd16)


_BD = 512


def _dc_body(mu_i_ref, mu_j_ref, r0_ref, r1_ref, dc_ref, dec_ref):
  j = pl.program_id(1)
  dc_ref[...] = lax.dot_general(
      mu_i_ref[...], mu_j_ref[...],
      dimension_numbers=(((1,), (1,)), ((), ())),
      preferred_element_type=jnp.float32)

  @pl.when(j == 0)
  def _():
    dec_ref[...] = (r0_ref[...] + r1_ref[...])[:, :1]


def _tc_dc(mu, r0, r1):
  g = pl.cdiv(N, _BD)
  return pl.pallas_call(
      _dc_body,
      grid=(g, g),
      in_specs=[
          pl.BlockSpec((_BD, 32), lambda i, j: (i, 0)),
          pl.BlockSpec((_BD, 32), lambda i, j: (j, 0)),
          pl.BlockSpec((_BD, 16), lambda i, j: (i, 0)),
          pl.BlockSpec((_BD, 16), lambda i, j: (i, 0)),
      ],
      out_specs=[
          pl.BlockSpec((_BD, _BD), lambda i, j: (i, j)),
          pl.BlockSpec((_BD, 1), lambda i, j: (i, 0)),
      ],
      out_shape=[
          jax.ShapeDtypeStruct((N, N), jnp.float32),
          jax.ShapeDtypeStruct((N, 1), jnp.float32),
      ],
  )(mu, r0, r1)


def kernel(x, edge_index, W1, W2, W3, Wd):
  zeros32 = jnp.zeros((N, 32), jnp.float32)
  zeros64 = jnp.zeros((N, 64), jnp.float32)
  zeros16 = jnp.zeros((N, 16), jnp.float32)
  w23 = jnp.concatenate([W2, W3], axis=1)
  wd16 = jnp.tile(Wd, (1, 16))

  t1 = _tc_mm1(x, W1)
  p = _sc_spmm32(t1, edge_index, zeros32)
  t23 = _tc_stage_b(p[0], p[1], w23)
  q = _sc_spmm64(t23, edge_index, zeros64)
  mu, logvar, d = _tc_stage_c(q[0], q[1], wd16)
  r = _sc_spmm16(d, edge_index, zeros16)
  dc, dec = _tc_dc(mu, r[0], r[1])
  return (dec, dc, mu, logvar)


# speedup vs baseline: 5.8848x; 5.8848x over previous
import functools

import jax
import jax.numpy as jnp
from jax import lax
from jax.experimental import pallas as pl
from jax.experimental.pallas import tpu as pltpu
from jax.experimental.pallas import tpu_sc as plsc

N = 10000
E = 320000
D = 128

NC = 2
NS = 16
NW = NC * NS
EPW = E // NW
CHUNK = 80
NCHUNK = EPW // CHUNK
STRIPE = 624


def _make_sc_spmm(width: int):
  mesh = plsc.VectorSubcoreMesh(core_axis_name="c", subcore_axis_name="s")

  @functools.partial(
      pl.kernel,
      out_type=jax.ShapeDtypeStruct((NC, N, width), jnp.float32),
      mesh=mesh,
      scratch_types=[
          pltpu.VMEM((CHUNK,), jnp.int32),
          pltpu.VMEM((CHUNK,), jnp.int32),
          pltpu.VMEM((CHUNK, width), jnp.float32),
          pltpu.VMEM_SHARED((N, width), jnp.float32),
          pltpu.SemaphoreType.DMA,
      ],
      compiler_params=pltpu.CompilerParams(use_tc_tiling_on_sc=False),
  )
  def spmm(table_hbm, edge_hbm, zeros_hbm, out_hbm, idx_src, idx_dst, rows,
           acc, sem):
    c = lax.axis_index("c")
    s = lax.axis_index("s")
    wid = s * NC + c
    stripe = pl.ds(s * STRIPE, STRIPE)
    tail = pl.ds(NS * STRIPE, N - NS * STRIPE)
    pltpu.sync_copy(zeros_hbm.at[stripe], acc.at[stripe])

    @pl.when(s == NS - 1)
    def _():
      pltpu.sync_copy(zeros_hbm.at[tail], acc.at[tail])

    plsc.subcore_barrier()

    base = wid * EPW

    def body(j, carry):
      off = base + j * CHUNK
      pltpu.sync_copy(edge_hbm.at[0, pl.ds(off, CHUNK)], idx_src)
      pltpu.sync_copy(edge_hbm.at[1, pl.ds(off, CHUNK)], idx_dst)
      pltpu.async_copy(table_hbm.at[idx_src], rows, sem).wait()
      pltpu.sync_copy(rows, acc.at[idx_dst], add=True)
      return carry

    lax.fori_loop(0, NCHUNK, body, 0)
    plsc.subcore_barrier()
    pltpu.sync_copy(acc.at[stripe], out_hbm.at[c, stripe])

    @pl.when(s == NS - 1)
    def _():
      pltpu.sync_copy(acc.at[tail], out_hbm.at[c, tail])

  return spmm


_sc_spmm32 = _make_sc_spmm(32)
_sc_spmm64 = _make_sc_spmm(64)
_sc_spmm16 = _make_sc_spmm(16)


_BN = 1000


def _mm1_body(x_ref, w_ref, o_ref):
  o_ref[...] = jnp.dot(x_ref[...], w_ref[...],
                       preferred_element_type=jnp.float32)


def _tc_mm1(x, w1):
  return pl.pallas_call(
      _mm1_body,
      grid=(N // _BN,),
      in_specs=[
          pl.BlockSpec((_BN, D), lambda i: (i, 0)),
          pl.BlockSpec((D, 32), lambda i: (0, 0)),
      ],
      out_specs=pl.BlockSpec((_BN, 32), lambda i: (i, 0)),
      out_shape=jax.ShapeDtypeStruct((N, 32), jnp.float32),
  )(x, w1)


def _stage_b_body(p0_ref, p1_ref, w_ref, o_ref):
  h = jnp.maximum(p0_ref[...] + p1_ref[...], 0.0)
  o_ref[...] = jnp.dot(h, w_ref[...], preferred_element_type=jnp.float32)


def _tc_stage_b(p0, p1, w23):
  return pl.pallas_call(
      _stage_b_body,
      grid=(N // _BN,),
      in_specs=[
          pl.BlockSpec((_BN, 32), lambda i: (i, 0)),
          pl.BlockSpec((_BN, 32), lambda i: (i, 0)),
          pl.BlockSpec((32, 64), lambda i: (0, 0)),
      ],
      out_specs=pl.BlockSpec((_BN, 64), lambda i: (i, 0)),
      out_shape=jax.ShapeDtypeStruct((N, 64), jnp.float32),
  )(p0, p1, w23)


def _stage_c_body(q0_ref, q1_ref, wd_ref, mu_ref, lv_ref, d_ref):
  sm = q0_ref[...] + q1_ref[...]
  mu = sm[:, :32]
  mu_ref[...] = mu
  lv_ref[...] = sm[:, 32:]
  d_ref[...] = jnp.dot(mu, wd_ref[...], preferred_element_type=jnp.float32)


def _tc_stage_c(q0, q1, wd16):
  return pl.pallas_call(
      _stage_c_body,
      grid=(N // _BN,),
      in_specs=[
          pl.BlockSpec((_BN, 64), lambda i: (i, 0)),
          pl.BlockSpec((_BN, 64), lambda i: (i, 0)),
          pl.BlockSpec((32, 16), lambda i: (0, 0)),
      ],
      out_specs=[
          pl.BlockSpec((_BN, 32), lambda i: (i, 0)),
          pl.BlockSpec((_BN, 32), lambda i: (i, 0)),
          pl.BlockSpec((_BN, 16), lambda i: (i, 0)),
      ],
      out_shape=[
          jax.ShapeDtypeStruct((N, 32), jnp.float32),
          jax.ShapeDtypeStruct((N, 32), jnp.float32),
          jax.ShapeDtypeStruct((N, 16), jnp.float32),
      ],
  )(q0, q1, wd16)


_BD = 512


def _dc_body(mu_i_ref, mu_j_ref, r0_ref, r1_ref, dc_ref, dec_ref):
  j = pl.program_id(1)
  dc_ref[...] = lax.dot_general(
      mu_i_ref[...], mu_j_ref[...],
      dimension_numbers=(((1,), (1,)), ((), ())),
      preferred_element_type=jnp.float32)

  @pl.when(j == 0)
  def _():
    dec_ref[...] = (r0_ref[...] + r1_ref[...])[:, :1]


def _tc_dc(mu, r0, r1):
  g = pl.cdiv(N, _BD)
  return pl.pallas_call(
      _dc_body,
      grid=(g, g),
      in_specs=[
          pl.BlockSpec((_BD, 32), lambda i, j: (i, 0)),
          pl.BlockSpec((_BD, 32), lambda i, j: (j, 0)),
          pl.BlockSpec((_BD, 16), lambda i, j: (i, 0)),
          pl.BlockSpec((_BD, 16), lambda i, j: (i, 0)),
      ],
      out_specs=[
          pl.BlockSpec((_BD, _BD), lambda i, j: (i, j)),
          pl.BlockSpec((_BD, 1), lambda i, j: (i, 0)),
      ],
      out_shape=[
          jax.ShapeDtypeStruct((N, N), jnp.float32),
          jax.ShapeDtypeStruct((N, 1), jnp.float32),
      ],
  )(mu, mu, r0, r1)


def kernel(x, edge_index, W1, W2, W3, Wd):
  zeros32 = jnp.zeros((N, 32), jnp.float32)
  zeros64 = jnp.zeros((N, 64), jnp.float32)
  zeros16 = jnp.zeros((N, 16), jnp.float32)
  w23 = jnp.concatenate([W2, W3], axis=1)
  wd16 = jnp.tile(Wd, (1, 16))

  t1 = _tc_mm1(x, W1)
  p = _sc_spmm32(t1, edge_index, zeros32)
  t23 = _tc_stage_b(p[0], p[1], w23)
  q = _sc_spmm64(t23, edge_index, zeros64)
  mu, logvar, d = _tc_stage_c(q[0], q[1], wd16)
  r = _sc_spmm16(d, edge_index, zeros16)
  dc, dec = _tc_dc(mu, r[0], r[1])
  return (dec, dc, mu, logvar)


# speedup vs baseline: 10.9763x; 1.8652x over previous
import functools

import jax
import jax.numpy as jnp
from jax import lax
from jax.experimental import pallas as pl
from jax.experimental.pallas import tpu as pltpu
from jax.experimental.pallas import tpu_sc as plsc

N = 10000
E = 320000
D = 128

NC = 2
NS = 16
NW = NC * NS
EPW = E // NW
CHUNK = 80
NCHUNK = EPW // CHUNK
STRIPE = 624


NB = 5
NGROUP = NCHUNK // NB


def _make_sc_spmm(width: int):
  mesh = plsc.VectorSubcoreMesh(core_axis_name="c", subcore_axis_name="s")

  @functools.partial(
      pl.kernel,
      out_type=jax.ShapeDtypeStruct((NC, N, width), jnp.float32),
      mesh=mesh,
      scratch_types=[
          pltpu.VMEM((NCHUNK, CHUNK), jnp.int32),
          pltpu.VMEM((NCHUNK, CHUNK), jnp.int32),
          pltpu.VMEM((NB, CHUNK, width), jnp.float32),
          pltpu.VMEM_SHARED((N, width), jnp.float32),
          pltpu.SemaphoreType.DMA((NB,)),
          pltpu.SemaphoreType.DMA((NB,)),
      ],
      compiler_params=pltpu.CompilerParams(use_tc_tiling_on_sc=False),
  )
  def spmm(table_hbm, src_hbm, dst_hbm, zeros_hbm, out_hbm, src_idx, dst_idx,
           rows, acc, gsem, ssem):
    c = lax.axis_index("c")
    s = lax.axis_index("s")
    wid = s * NC + c
    pltpu.sync_copy(src_hbm.at[wid], src_idx)
    pltpu.sync_copy(dst_hbm.at[wid], dst_idx)
    for b in range(NB):
      pltpu.async_copy(table_hbm.at[src_idx.at[b]], rows.at[b], gsem.at[b])
    stripe = pl.ds(s * STRIPE, STRIPE)
    tail = pl.ds(NS * STRIPE, N - NS * STRIPE)
    pltpu.sync_copy(zeros_hbm.at[stripe], acc.at[stripe])

    @pl.when(s == NS - 1)
    def _():
      pltpu.sync_copy(zeros_hbm.at[tail], acc.at[tail])

    plsc.subcore_barrier()

    def body(g, carry):
      for b in range(NB):
        j = g * NB + b
        pltpu.make_async_copy(table_hbm.at[src_idx.at[b]], rows.at[b],
                              gsem.at[b]).wait()
        pltpu.async_copy(rows.at[b], acc.at[dst_idx.at[j]], ssem.at[b],
                         add=True)
      for b in range(NB):
        j = g * NB + b
        pltpu.make_async_copy(rows.at[b], acc.at[dst_idx.at[j]],
                              ssem.at[b]).wait()

        @pl.when(j + NB < NCHUNK)
        def _():
          pltpu.async_copy(table_hbm.at[src_idx.at[j + NB]], rows.at[b],
                           gsem.at[b])

      return carry

    lax.fori_loop(0, NGROUP, body, 0)
    plsc.subcore_barrier()
    pltpu.sync_copy(acc.at[stripe], out_hbm.at[c, stripe])

    @pl.when(s == NS - 1)
    def _():
      pltpu.sync_copy(acc.at[tail], out_hbm.at[c, tail])

  return spmm


_sc_spmm32 = _make_sc_spmm(32)
_sc_spmm64 = _make_sc_spmm(64)
_sc_spmm16 = _make_sc_spmm(16)


_BN = 1000


def _mm1_body(x_ref, w_ref, o_ref):
  o_ref[...] = jnp.dot(x_ref[...], w_ref[...],
                       preferred_element_type=jnp.float32)


def _tc_mm1(x, w1):
  return pl.pallas_call(
      _mm1_body,
      grid=(N // _BN,),
      in_specs=[
          pl.BlockSpec((_BN, D), lambda i: (i, 0)),
          pl.BlockSpec((D, 32), lambda i: (0, 0)),
      ],
      out_specs=pl.BlockSpec((_BN, 32), lambda i: (i, 0)),
      out_shape=jax.ShapeDtypeStruct((N, 32), jnp.float32),
  )(x, w1)


def _stage_b_body(p0_ref, p1_ref, w_ref, o_ref):
  h = jnp.maximum(p0_ref[...] + p1_ref[...], 0.0)
  o_ref[...] = jnp.dot(h, w_ref[...], preferred_element_type=jnp.float32)


def _tc_stage_b(p0, p1, w23):
  return pl.pallas_call(
      _stage_b_body,
      grid=(N // _BN,),
      in_specs=[
          pl.BlockSpec((_BN, 32), lambda i: (i, 0)),
          pl.BlockSpec((_BN, 32), lambda i: (i, 0)),
          pl.BlockSpec((32, 64), lambda i: (0, 0)),
      ],
      out_specs=pl.BlockSpec((_BN, 64), lambda i: (i, 0)),
      out_shape=jax.ShapeDtypeStruct((N, 64), jnp.float32),
  )(p0, p1, w23)


def _stage_c_body(q0_ref, q1_ref, wd_ref, mu_ref, lv_ref, d_ref):
  sm = q0_ref[...] + q1_ref[...]
  mu = sm[:, :32]
  mu_ref[...] = mu
  lv_ref[...] = sm[:, 32:]
  d_ref[...] = jnp.dot(mu, wd_ref[...], preferred_element_type=jnp.float32)


def _tc_stage_c(q0, q1, wd16):
  return pl.pallas_call(
      _stage_c_body,
      grid=(N // _BN,),
      in_specs=[
          pl.BlockSpec((_BN, 64), lambda i: (i, 0)),
          pl.BlockSpec((_BN, 64), lambda i: (i, 0)),
          pl.BlockSpec((32, 16), lambda i: (0, 0)),
      ],
      out_specs=[
          pl.BlockSpec((_BN, 32), lambda i: (i, 0)),
          pl.BlockSpec((_BN, 32), lambda i: (i, 0)),
          pl.BlockSpec((_BN, 16), lambda i: (i, 0)),
      ],
      out_shape=[
          jax.ShapeDtypeStruct((N, 32), jnp.float32),
          jax.ShapeDtypeStruct((N, 32), jnp.float32),
          jax.ShapeDtypeStruct((N, 16), jnp.float32),
      ],
  )(q0, q1, wd16)


_BD = 512


def _dc_body(mu_i_ref, mu_j_ref, r0_ref, r1_ref, dc_ref, dec_ref):
  j = pl.program_id(1)
  dc_ref[...] = lax.dot_general(
      mu_i_ref[...], mu_j_ref[...],
      dimension_numbers=(((1,), (1,)), ((), ())),
      preferred_element_type=jnp.float32)

  @pl.when(j == 0)
  def _():
    dec_ref[...] = (r0_ref[...] + r1_ref[...])[:, :1]


def _tc_dc(mu, r0, r1):
  g = pl.cdiv(N, _BD)
  return pl.pallas_call(
      _dc_body,
      grid=(g, g),
      in_specs=[
          pl.BlockSpec((_BD, 32), lambda i, j: (i, 0)),
          pl.BlockSpec((_BD, 32), lambda i, j: (j, 0)),
          pl.BlockSpec((_BD, 16), lambda i, j: (i, 0)),
          pl.BlockSpec((_BD, 16), lambda i, j: (i, 0)),
      ],
      out_specs=[
          pl.BlockSpec((_BD, _BD), lambda i, j: (i, j)),
          pl.BlockSpec((_BD, 1), lambda i, j: (i, 0)),
      ],
      out_shape=[
          jax.ShapeDtypeStruct((N, N), jnp.float32),
          jax.ShapeDtypeStruct((N, 1), jnp.float32),
      ],
  )(mu, mu, r0, r1)


def kernel(x, edge_index, W1, W2, W3, Wd):
  zeros32 = jnp.zeros((N, 32), jnp.float32)
  zeros64 = jnp.zeros((N, 64), jnp.float32)
  zeros16 = jnp.zeros((N, 16), jnp.float32)
  w23 = jnp.concatenate([W2, W3], axis=1)
  wd16 = jnp.tile(Wd, (1, 16))
  src_r = edge_index[0].reshape(NW, NCHUNK, CHUNK)
  dst_r = edge_index[1].reshape(NW, NCHUNK, CHUNK)

  t1 = _tc_mm1(x, W1)
  p = _sc_spmm32(t1, src_r, dst_r, zeros32)
  t23 = _tc_stage_b(p[0], p[1], w23)
  q = _sc_spmm64(t23, src_r, dst_r, zeros64)
  mu, logvar, d = _tc_stage_c(q[0], q[1], wd16)
  r = _sc_spmm16(d, src_r, dst_r, zeros16)
  dc, dec = _tc_dc(mu, r[0], r[1])
  return (dec, dc, mu, logvar)


# speedup vs baseline: 14.5084x; 1.3218x over previous
import functools

import jax
import jax.numpy as jnp
from jax import lax
from jax.experimental import pallas as pl
from jax.experimental.pallas import tpu as pltpu
from jax.experimental.pallas import tpu_sc as plsc

N = 10000
E = 320000
D = 128

NC = 2
NS = 16
NW = NC * NS
EPW = E // NW
CHUNK = 80
NCHUNK = EPW // CHUNK
STRIPE = 624


NB = 5
NGROUP = NCHUNK // NB


def _make_sc_spmm(width: int):
  mesh = plsc.VectorSubcoreMesh(core_axis_name="c", subcore_axis_name="s")

  @functools.partial(
      pl.kernel,
      out_type=jax.ShapeDtypeStruct((NC, N, width), jnp.float32),
      mesh=mesh,
      scratch_types=[
          pltpu.VMEM((NCHUNK, CHUNK), jnp.int32),
          pltpu.VMEM((NCHUNK, CHUNK), jnp.int32),
          pltpu.VMEM((NB, CHUNK, width), jnp.float32),
          pltpu.VMEM_SHARED((N, width), jnp.float32),
          pltpu.SemaphoreType.DMA((NB,)),
          pltpu.SemaphoreType.DMA((NB,)),
      ],
      compiler_params=pltpu.CompilerParams(use_tc_tiling_on_sc=False),
  )
  def spmm(table_hbm, src_hbm, dst_hbm, zeros_hbm, out_hbm, src_idx, dst_idx,
           rows, acc, gsem, ssem):
    c = lax.axis_index("c")
    s = lax.axis_index("s")
    wid = s * NC + c
    pltpu.sync_copy(src_hbm.at[wid], src_idx)
    pltpu.sync_copy(dst_hbm.at[wid], dst_idx)
    for b in range(NB):
      pltpu.async_copy(table_hbm.at[src_idx.at[b]], rows.at[b], gsem.at[b])
    stripe = pl.ds(s * STRIPE, STRIPE)
    tail = pl.ds(NS * STRIPE, N - NS * STRIPE)
    pltpu.sync_copy(zeros_hbm.at[stripe], acc.at[stripe])

    @pl.when(s == NS - 1)
    def _():
      pltpu.sync_copy(zeros_hbm.at[tail], acc.at[tail])

    plsc.subcore_barrier()

    def body(g, carry):
      for b in range(NB):
        j = g * NB + b
        pltpu.make_async_copy(table_hbm.at[src_idx.at[b]], rows.at[b],
                              gsem.at[b]).wait()
        pltpu.async_copy(rows.at[b], acc.at[dst_idx.at[j]], ssem.at[b],
                         add=True)
      for b in range(NB):
        j = g * NB + b
        pltpu.make_async_copy(rows.at[b], acc.at[dst_idx.at[j]],
                              ssem.at[b]).wait()

        @pl.when(j + NB < NCHUNK)
        def _():
          pltpu.async_copy(table_hbm.at[src_idx.at[j + NB]], rows.at[b],
                           gsem.at[b])

      return carry

    lax.fori_loop(0, NGROUP, body, 0)
    plsc.subcore_barrier()
    pltpu.sync_copy(acc.at[stripe], out_hbm.at[c, stripe])

    @pl.when(s == NS - 1)
    def _():
      pltpu.sync_copy(acc.at[tail], out_hbm.at[c, tail])

  return spmm


_sc_spmm32 = _make_sc_spmm(32)
_sc_spmm64 = _make_sc_spmm(64)
_sc_spmm16 = _make_sc_spmm(16)


_BN = 1000


def _mm1_body(x_ref, w_ref, o_ref):
  o_ref[...] = jnp.dot(x_ref[...], w_ref[...],
                       preferred_element_type=jnp.float32)


def _tc_mm1(x, w1):
  return pl.pallas_call(
      _mm1_body,
      grid=(N // _BN,),
      in_specs=[
          pl.BlockSpec((_BN, D), lambda i: (i, 0)),
          pl.BlockSpec((D, 32), lambda i: (0, 0)),
      ],
      out_specs=pl.BlockSpec((_BN, 32), lambda i: (i, 0)),
      out_shape=jax.ShapeDtypeStruct((N, 32), jnp.float32),
  )(x, w1)


def _stage_b_body(p0_ref, p1_ref, w_ref, o_ref):
  h = jnp.maximum(p0_ref[...] + p1_ref[...], 0.0)
  o_ref[...] = jnp.dot(h, w_ref[...], preferred_element_type=jnp.float32)


def _tc_stage_b(p0, p1, w23):
  return pl.pallas_call(
      _stage_b_body,
      grid=(N // _BN,),
      in_specs=[
          pl.BlockSpec((_BN, 32), lambda i: (i, 0)),
          pl.BlockSpec((_BN, 32), lambda i: (i, 0)),
          pl.BlockSpec((32, 64), lambda i: (0, 0)),
      ],
      out_specs=pl.BlockSpec((_BN, 64), lambda i: (i, 0)),
      out_shape=jax.ShapeDtypeStruct((N, 64), jnp.float32),
  )(p0, p1, w23)


def _stage_c_body(q0_ref, q1_ref, wd_ref, mu_ref, mub_ref, lv_ref, d_ref):
  sm = q0_ref[...] + q1_ref[...]
  mu = sm[:, :32]
  mu_ref[...] = mu
  mub_ref[...] = mu.astype(jnp.bfloat16)
  lv_ref[...] = sm[:, 32:]
  d_ref[...] = jnp.dot(mu, wd_ref[...], preferred_element_type=jnp.float32)


def _tc_stage_c(q0, q1, wd16):
  return pl.pallas_call(
      _stage_c_body,
      grid=(N // _BN,),
      in_specs=[
          pl.BlockSpec((_BN, 64), lambda i: (i, 0)),
          pl.BlockSpec((_BN, 64), lambda i: (i, 0)),
          pl.BlockSpec((32, 16), lambda i: (0, 0)),
      ],
      out_specs=[
          pl.BlockSpec((_BN, 32), lambda i: (i, 0)),
          pl.BlockSpec((_BN, 32), lambda i: (i, 0)),
          pl.BlockSpec((_BN, 32), lambda i: (i, 0)),
          pl.BlockSpec((_BN, 16), lambda i: (i, 0)),
      ],
      out_shape=[
          jax.ShapeDtypeStruct((N, 32), jnp.float32),
          jax.ShapeDtypeStruct((N, 32), jnp.bfloat16),
          jax.ShapeDtypeStruct((N, 32), jnp.float32),
          jax.ShapeDtypeStruct((N, 16), jnp.float32),
      ],
  )(q0, q1, wd16)


_BDI = 512
_BDJ = 1024


def _dc_body(mu_i_ref, mu_j_ref, dc_ref):
  dc_ref[...] = lax.dot_general(
      mu_i_ref[...], mu_j_ref[...],
      dimension_numbers=(((1,), (1,)), ((), ())),
      preferred_element_type=jnp.float32)


def _tc_dc(mu_b):
  return pl.pallas_call(
      _dc_body,
      grid=(pl.cdiv(N, _BDI), pl.cdiv(N, _BDJ)),
      in_specs=[
          pl.BlockSpec((_BDI, 32), lambda i, j: (i, 0)),
          pl.BlockSpec((_BDJ, 32), lambda i, j: (j, 0)),
      ],
      out_specs=pl.BlockSpec((_BDI, _BDJ), lambda i, j: (i, j)),
      out_shape=jax.ShapeDtypeStruct((N, N), jnp.float32),
  )(mu_b, mu_b)


def _dec_body(r0_ref, r1_ref, dec_ref):
  dec_ref[...] = (r0_ref[...] + r1_ref[...])[:, :1]


def _tc_dec(r0, r1):
  return pl.pallas_call(
      _dec_body,
      in_specs=[
          pl.BlockSpec((N, 16), lambda: (0, 0)),
          pl.BlockSpec((N, 16), lambda: (0, 0)),
      ],
      out_specs=pl.BlockSpec((N, 1), lambda: (0, 0)),
      out_shape=jax.ShapeDtypeStruct((N, 1), jnp.float32),
  )(r0, r1)


def kernel(x, edge_index, W1, W2, W3, Wd):
  zeros32 = jnp.zeros((N, 32), jnp.float32)
  zeros64 = jnp.zeros((N, 64), jnp.float32)
  zeros16 = jnp.zeros((N, 16), jnp.float32)
  w23 = jnp.concatenate([W2, W3], axis=1)
  wd16 = jnp.tile(Wd, (1, 16))
  src_r = edge_index[0].reshape(NW, NCHUNK, CHUNK)
  dst_r = edge_index[1].reshape(NW, NCHUNK, CHUNK)

  t1 = _tc_mm1(x, W1)
  p = _sc_spmm32(t1, src_r, dst_r, zeros32)
  t23 = _tc_stage_b(p[0], p[1], w23)
  q = _sc_spmm64(t23, src_r, dst_r, zeros64)
  mu, mu_b, logvar, d = _tc_stage_c(q[0], q[1], wd16)
  r = _sc_spmm16(d, src_r, dst_r, zeros16)
  dc = _tc_dc(mu_b)
  dec = _tc_dec(r[0], r[1])
  return (dec, dc, mu, logvar)


# speedup vs baseline: 14.6766x; 1.0116x over previous
import functools

import jax
import jax.numpy as jnp
from jax import lax
from jax.experimental import pallas as pl
from jax.experimental.pallas import tpu as pltpu
from jax.experimental.pallas import tpu_sc as plsc

N = 10000
E = 320000
D = 128

NC = 2
NS = 16
NW = NC * NS
EPW = E // NW
CHUNK = 80
NCHUNK = EPW // CHUNK
STRIPE = 624
NB = 5
NGROUP = NCHUNK // NB

_SC_PARAMS = pltpu.CompilerParams(use_tc_tiling_on_sc=False)


def _edge_prolog(edge_hbm, src_idx, dst_idx, wid):
  sl = pl.ds(wid * EPW, EPW)
  pltpu.sync_copy(edge_hbm.at[0, sl], src_idx)
  pltpu.sync_copy(edge_hbm.at[1, sl], dst_idx)


def _zero_acc(zeros_hbm, acc, s):
  stripe = pl.ds(s * STRIPE, STRIPE)
  tail = pl.ds(NS * STRIPE, N - NS * STRIPE)
  pltpu.sync_copy(zeros_hbm.at[stripe], acc.at[stripe])

  @pl.when(s == NS - 1)
  def _():
    pltpu.sync_copy(zeros_hbm.at[tail], acc.at[tail])


def _writeout(acc, out_hbm, c, s):
  stripe = pl.ds(s * STRIPE, STRIPE)
  tail = pl.ds(NS * STRIPE, N - NS * STRIPE)
  pltpu.sync_copy(acc.at[stripe], out_hbm.at[c, stripe])

  @pl.when(s == NS - 1)
  def _():
    pltpu.sync_copy(acc.at[tail], out_hbm.at[c, tail])


def _make_sc_spmm(width: int):
  mesh = plsc.VectorSubcoreMesh(core_axis_name="c", subcore_axis_name="s")

  @functools.partial(
      pl.kernel,
      out_type=jax.ShapeDtypeStruct((NC, N, width), jnp.float32),
      mesh=mesh,
      scratch_types=[
          pltpu.VMEM((EPW,), jnp.int32),
          pltpu.VMEM((EPW,), jnp.int32),
          pltpu.VMEM((NB, CHUNK, width), jnp.float32),
          pltpu.VMEM_SHARED((N, width), jnp.float32),
          pltpu.SemaphoreType.DMA((NB,)),
          pltpu.SemaphoreType.DMA((NB,)),
      ],
      compiler_params=_SC_PARAMS,
  )
  def spmm(table_hbm, edge_hbm, zeros_hbm, out_hbm, src_idx, dst_idx,
           rows, acc, gsem, ssem):
    c = lax.axis_index("c")
    s = lax.axis_index("s")
    wid = s * NC + c
    _edge_prolog(edge_hbm, src_idx, dst_idx, wid)

    def sidx(j):
      return src_idx.at[pl.ds(j * CHUNK, CHUNK)]

    def didx(j):
      return dst_idx.at[pl.ds(j * CHUNK, CHUNK)]

    for b in range(NB):
      pltpu.async_copy(table_hbm.at[sidx(b)], rows.at[b], gsem.at[b])
    _zero_acc(zeros_hbm, acc, s)
    plsc.subcore_barrier()

    def body(g, carry):
      for b in range(NB):
        j = g * NB + b
        pltpu.make_async_copy(table_hbm.at[sidx(j)], rows.at[b],
                              gsem.at[b]).wait()
        pltpu.async_copy(rows.at[b], acc.at[didx(j)], ssem.at[b], add=True)
      for b in range(NB):
        j = g * NB + b
        pltpu.make_async_copy(rows.at[b], acc.at[didx(j)], ssem.at[b]).wait()

        @pl.when(j + NB < NCHUNK)
        def _():
          pltpu.async_copy(table_hbm.at[sidx(j + NB)], rows.at[b], gsem.at[b])

      return carry

    lax.fori_loop(0, NGROUP, body, 0)
    plsc.subcore_barrier()
    _writeout(acc, out_hbm, c, s)

  return spmm


_sc_spmm32 = _make_sc_spmm(32)


def _make_sc_spmm_dual(wa: int, wb: int):
  mesh = plsc.VectorSubcoreMesh(core_axis_name="c", subcore_axis_name="s")

  @functools.partial(
      pl.kernel,
      out_type=[
          jax.ShapeDtypeStruct((NC, N, wa), jnp.float32),
          jax.ShapeDtypeStruct((NC, N, wb), jnp.float32),
      ],
      mesh=mesh,
      scratch_types=[
          pltpu.VMEM((EPW,), jnp.int32),
          pltpu.VMEM((EPW,), jnp.int32),
          pltpu.VMEM((NB, CHUNK, wa), jnp.float32),
          pltpu.VMEM((NB, CHUNK, wb), jnp.float32),
          pltpu.VMEM_SHARED((N, wa), jnp.float32),
          pltpu.VMEM_SHARED((N, wb), jnp.float32),
          pltpu.SemaphoreType.DMA((NB,)),
          pltpu.SemaphoreType.DMA((NB,)),
          pltpu.SemaphoreType.DMA((NB,)),
          pltpu.SemaphoreType.DMA((NB,)),
      ],
      compiler_params=_SC_PARAMS,
  )
  def spmm2(ta_hbm, tb_hbm, edge_hbm, za_hbm, zb_hbm, outa_hbm, outb_hbm,
            src_idx, dst_idx, rows_a, rows_b, acc_a, acc_b,
            gsa, gsb, ssa, ssb):
    c = lax.axis_index("c")
    s = lax.axis_index("s")
    wid = s * NC + c
    _edge_prolog(edge_hbm, src_idx, dst_idx, wid)

    def sidx(j):
      return src_idx.at[pl.ds(j * CHUNK, CHUNK)]

    def didx(j):
      return dst_idx.at[pl.ds(j * CHUNK, CHUNK)]

    for b in range(NB):
      pltpu.async_copy(ta_hbm.at[sidx(b)], rows_a.at[b], gsa.at[b])
      pltpu.async_copy(tb_hbm.at[sidx(b)], rows_b.at[b], gsb.at[b])
    _zero_acc(za_hbm, acc_a, s)
    _zero_acc(zb_hbm, acc_b, s)
    plsc.subcore_barrier()

    def body(g, carry):
      for b in range(NB):
        j = g * NB + b
        pltpu.make_async_copy(ta_hbm.at[sidx(j)], rows_a.at[b],
                              gsa.at[b]).wait()
        pltpu.async_copy(rows_a.at[b], acc_a.at[didx(j)], ssa.at[b], add=True)
        pltpu.make_async_copy(tb_hbm.at[sidx(j)], rows_b.at[b],
                              gsb.at[b]).wait()
        pltpu.async_copy(rows_b.at[b], acc_b.at[didx(j)], ssb.at[b], add=True)
      for b in range(NB):
        j = g * NB + b
        pltpu.make_async_copy(rows_a.at[b], acc_a.at[didx(j)],
                              ssa.at[b]).wait()
        pltpu.make_async_copy(rows_b.at[b], acc_b.at[didx(j)],
                              ssb.at[b]).wait()

        @pl.when(j + NB < NCHUNK)
        def _():
          pltpu.async_copy(ta_hbm.at[sidx(j + NB)], rows_a.at[b], gsa.at[b])
          pltpu.async_copy(tb_hbm.at[sidx(j + NB)], rows_b.at[b], gsb.at[b])

      return carry

    lax.fori_loop(0, NGROUP, body, 0)
    plsc.subcore_barrier()
    _writeout(acc_a, outa_hbm, c, s)
    _writeout(acc_b, outb_hbm, c, s)

  return spmm2


_sc_spmm_dual = _make_sc_spmm_dual(32, 16)


_BN = 1000


def _mm1_body(x_ref, w_ref, o_ref):
  o_ref[...] = jnp.dot(x_ref[...], w_ref[...],
                       preferred_element_type=jnp.float32)


def _tc_mm1(x, w1):
  return pl.pallas_call(
      _mm1_body,
      grid=(N // _BN,),
      in_specs=[
          pl.BlockSpec((_BN, D), lambda i: (i, 0)),
          pl.BlockSpec((D, 32), lambda i: (0, 0)),
      ],
      out_specs=pl.BlockSpec((_BN, 32), lambda i: (i, 0)),
      out_shape=jax.ShapeDtypeStruct((N, 32), jnp.float32),
  )(x, w1)


def _stage_b_body(p0_ref, p1_ref, w2_ref, w3_ref, t2_ref, t3_ref):
  h = jnp.maximum(p0_ref[...] + p1_ref[...], 0.0)
  t2_ref[...] = jnp.dot(h, w2_ref[...], preferred_element_type=jnp.float32)
  t3_ref[...] = jnp.dot(h, w3_ref[...], preferred_element_type=jnp.float32)


def _tc_stage_b(p0, p1, w2, w3):
  return pl.pallas_call(
      _stage_b_body,
      grid=(N // _BN,),
      in_specs=[
          pl.BlockSpec((_BN, 32), lambda i: (i, 0)),
          pl.BlockSpec((_BN, 32), lambda i: (i, 0)),
          pl.BlockSpec((32, 32), lambda i: (0, 0)),
          pl.BlockSpec((32, 32), lambda i: (0, 0)),
      ],
      out_specs=[
          pl.BlockSpec((_BN, 32), lambda i: (i, 0)),
          pl.BlockSpec((_BN, 32), lambda i: (i, 0)),
      ],
      out_shape=[
          jax.ShapeDtypeStruct((N, 32), jnp.float32),
          jax.ShapeDtypeStruct((N, 32), jnp.float32),
      ],
  )(p0, p1, w2, w3)


def _stage_c_body(q0_ref, q1_ref, wd_ref, mu_ref, mub_ref, d_ref):
  mu = q0_ref[...] + q1_ref[...]
  mu_ref[...] = mu
  mub_ref[...] = mu.astype(jnp.bfloat16)
  d_ref[...] = jnp.dot(mu, wd_ref[...], preferred_element_type=jnp.float32)


def _tc_stage_c(q0, q1, wd16):
  return pl.pallas_call(
      _stage_c_body,
      grid=(N // _BN,),
      in_specs=[
          pl.BlockSpec((_BN, 32), lambda i: (i, 0)),
          pl.BlockSpec((_BN, 32), lambda i: (i, 0)),
          pl.BlockSpec((32, 16), lambda i: (0, 0)),
      ],
      out_specs=[
          pl.BlockSpec((_BN, 32), lambda i: (i, 0)),
          pl.BlockSpec((_BN, 32), lambda i: (i, 0)),
          pl.BlockSpec((_BN, 16), lambda i: (i, 0)),
      ],
      out_shape=[
          jax.ShapeDtypeStruct((N, 32), jnp.float32),
          jax.ShapeDtypeStruct((N, 32), jnp.bfloat16),
          jax.ShapeDtypeStruct((N, 16), jnp.float32),
      ],
  )(q0, q1, wd16)


_BDI = 512
_BDJ = 1024


def _dc_body(mu_i_ref, mu_j_ref, dc_ref):
  dc_ref[...] = lax.dot_general(
      mu_i_ref[...], mu_j_ref[...],
      dimension_numbers=(((1,), (1,)), ((), ())),
      preferred_element_type=jnp.float32)


def _tc_dc(mu_b):
  return pl.pallas_call(
      _dc_body,
      grid=(pl.cdiv(N, _BDI), pl.cdiv(N, _BDJ)),
      in_specs=[
          pl.BlockSpec((_BDI, 32), lambda i, j: (i, 0)),
          pl.BlockSpec((_BDJ, 32), lambda i, j: (j, 0)),
      ],
      out_specs=pl.BlockSpec((_BDI, _BDJ), lambda i, j: (i, j)),
      out_shape=jax.ShapeDtypeStruct((N, N), jnp.float32),
  )(mu_b, mu_b)


def _final_body(lv0_ref, lv1_ref, r0_ref, r1_ref, lv_ref, dec_ref):
  lv_ref[...] = lv0_ref[...] + lv1_ref[...]
  dec_ref[...] = (r0_ref[...] + r1_ref[...])[:, :1]


def _tc_final(lv0, lv1, r0, r1):
  return pl.pallas_call(
      _final_body,
      in_specs=[
          pl.BlockSpec((N, 32), lambda: (0, 0)),
          pl.BlockSpec((N, 32), lambda: (0, 0)),
          pl.BlockSpec((N, 16), lambda: (0, 0)),
          pl.BlockSpec((N, 16), lambda: (0, 0)),
      ],
      out_specs=[
          pl.BlockSpec((N, 32), lambda: (0, 0)),
          pl.BlockSpec((N, 1), lambda: (0, 0)),
      ],
      out_shape=[
          jax.ShapeDtypeStruct((N, 32), jnp.float32),
          jax.ShapeDtypeStruct((N, 1), jnp.float32),
      ],
  )(lv0, lv1, r0, r1)


def kernel(x, edge_index, W1, W2, W3, Wd):
  zeros32 = jnp.zeros((N, 32), jnp.float32)
  zeros16 = jnp.zeros((N, 16), jnp.float32)
  wd16 = jnp.tile(Wd, (1, 16))

  t1 = _tc_mm1(x, W1)
  p = _sc_spmm32(t1, edge_index, zeros32)
  t2, t3 = _tc_stage_b(p[0], p[1], W2, W3)
  q = _sc_spmm32(t2, edge_index, zeros32)
  mu, mu_b, d = _tc_stage_c(q[0], q[1], wd16)
  lv, r = _sc_spmm_dual(t3, d, edge_index, zeros32, zeros16)
  dc = _tc_dc(mu_b)
  logvar, dec = _tc_final(lv[0], lv[1], r[0], r[1])
  return (dec, dc, mu, logvar)


# speedup vs baseline: 16.2643x; 1.1082x over previous
import functools

import jax
import jax.numpy as jnp
from jax import lax
from jax.experimental import pallas as pl
from jax.experimental.pallas import tpu as pltpu
from jax.experimental.pallas import tpu_sc as plsc

N = 10000
E = 320000
D = 128

NC = 2
NS = 16
NW = NC * NS
EPW = E // NW
CHUNK = 80
NCHUNK = EPW // CHUNK
STRIPE = 624

_SC_PARAMS = pltpu.CompilerParams(use_tc_tiling_on_sc=False)


def _edge_prolog(edge_hbm, src_idx, dst_idx, wid):
  sl = pl.ds(wid * EPW, EPW)
  pltpu.sync_copy(edge_hbm.at[0, sl], src_idx)
  pltpu.sync_copy(edge_hbm.at[1, sl], dst_idx)


def _zero_acc(zeros_hbm, acc, s):
  stripe = pl.ds(s * STRIPE, STRIPE)
  tail = pl.ds(NS * STRIPE, N - NS * STRIPE)
  pltpu.sync_copy(zeros_hbm.at[stripe], acc.at[stripe])

  @pl.when(s == NS - 1)
  def _():
    pltpu.sync_copy(zeros_hbm.at[tail], acc.at[tail])


def _writeout(acc, out_hbm, c, s):
  stripe = pl.ds(s * STRIPE, STRIPE)
  tail = pl.ds(NS * STRIPE, N - NS * STRIPE)
  pltpu.sync_copy(acc.at[stripe], out_hbm.at[c, stripe])

  @pl.when(s == NS - 1)
  def _():
    pltpu.sync_copy(acc.at[tail], out_hbm.at[c, tail])


def _make_sc_spmm(width: int, nb: int):
  mesh = plsc.VectorSubcoreMesh(core_axis_name="c", subcore_axis_name="s")

  @functools.partial(
      pl.kernel,
      out_type=jax.ShapeDtypeStruct((NC, N, width), jnp.float32),
      mesh=mesh,
      scratch_types=[
          pltpu.VMEM((EPW,), jnp.int32),
          pltpu.VMEM((EPW,), jnp.int32),
          pltpu.VMEM((nb, CHUNK, width), jnp.float32),
          pltpu.VMEM_SHARED((N, width), jnp.float32),
          pltpu.SemaphoreType.DMA((nb,)),
          pltpu.SemaphoreType.DMA((nb,)),
      ],
      compiler_params=_SC_PARAMS,
  )
  def spmm(table_hbm, edge_hbm, zeros_hbm, out_hbm, src_idx, dst_idx,
           rows, acc, gsem, ssem):
    c = lax.axis_index("c")
    s = lax.axis_index("s")
    wid = s * NC + c
    _edge_prolog(edge_hbm, src_idx, dst_idx, wid)

    def sidx(j):
      return src_idx.at[pl.ds(j * CHUNK, CHUNK)]

    def didx(j):
      return dst_idx.at[pl.ds(j * CHUNK, CHUNK)]

    for b in range(nb):
      pltpu.async_copy(table_hbm.at[sidx(b)], rows.at[b], gsem.at[b])
    _zero_acc(zeros_hbm, acc, s)
    plsc.subcore_barrier()

    def body(g, carry):
      for b in range(nb):
        j = g * nb + b
        pltpu.make_async_copy(table_hbm.at[sidx(j)], rows.at[b],
                              gsem.at[b]).wait()
        pltpu.async_copy(rows.at[b], acc.at[didx(j)], ssem.at[b], add=True)
      for b in range(nb):
        j = g * nb + b
        pltpu.make_async_copy(rows.at[b], acc.at[didx(j)], ssem.at[b]).wait()

        @pl.when(j + nb < NCHUNK)
        def _():
          pltpu.async_copy(table_hbm.at[sidx(j + nb)], rows.at[b], gsem.at[b])

      return carry

    lax.fori_loop(0, NCHUNK // nb, body, 0)
    plsc.subcore_barrier()
    _writeout(acc, out_hbm, c, s)

  return spmm


_sc_spmm32 = _make_sc_spmm(32, 5)


def _make_sc_spmm_dual(wa: int, wb: int, nb: int):
  mesh = plsc.VectorSubcoreMesh(core_axis_name="c", subcore_axis_name="s")

  @functools.partial(
      pl.kernel,
      out_type=[
          jax.ShapeDtypeStruct((NC, N, wa), jnp.float32),
          jax.ShapeDtypeStruct((NC, N, wb), jnp.float32),
      ],
      mesh=mesh,
      scratch_types=[
          pltpu.VMEM((EPW,), jnp.int32),
          pltpu.VMEM((EPW,), jnp.int32),
          pltpu.VMEM((nb, CHUNK, wa), jnp.float32),
          pltpu.VMEM((nb, CHUNK, wb), jnp.float32),
          pltpu.VMEM_SHARED((N, wa), jnp.float32),
          pltpu.VMEM_SHARED((N, wb), jnp.float32),
          pltpu.SemaphoreType.DMA((nb,)),
          pltpu.SemaphoreType.DMA((nb,)),
          pltpu.SemaphoreType.DMA((nb,)),
          pltpu.SemaphoreType.DMA((nb,)),
      ],
      compiler_params=_SC_PARAMS,
  )
  def spmm2(ta_hbm, tb_hbm, edge_hbm, za_hbm, zb_hbm, outa_hbm, outb_hbm,
            src_idx, dst_idx, rows_a, rows_b, acc_a, acc_b,
            gsa, gsb, ssa, ssb):
    c = lax.axis_index("c")
    s = lax.axis_index("s")
    wid = s * NC + c
    _edge_prolog(edge_hbm, src_idx, dst_idx, wid)

    def sidx(j):
      return src_idx.at[pl.ds(j * CHUNK, CHUNK)]

    def didx(j):
      return dst_idx.at[pl.ds(j * CHUNK, CHUNK)]

    for b in range(nb):
      pltpu.async_copy(ta_hbm.at[sidx(b)], rows_a.at[b], gsa.at[b])
      pltpu.async_copy(tb_hbm.at[sidx(b)], rows_b.at[b], gsb.at[b])
    _zero_acc(za_hbm, acc_a, s)
    _zero_acc(zb_hbm, acc_b, s)
    plsc.subcore_barrier()

    def body(g, carry):
      for b in range(nb):
        j = g * nb + b
        pltpu.make_async_copy(ta_hbm.at[sidx(j)], rows_a.at[b],
                              gsa.at[b]).wait()
        pltpu.async_copy(rows_a.at[b], acc_a.at[didx(j)], ssa.at[b], add=True)
        pltpu.make_async_copy(tb_hbm.at[sidx(j)], rows_b.at[b],
                              gsb.at[b]).wait()
        pltpu.async_copy(rows_b.at[b], acc_b.at[didx(j)], ssb.at[b], add=True)
      for b in range(nb):
        j = g * nb + b
        pltpu.make_async_copy(rows_a.at[b], acc_a.at[didx(j)],
                              ssa.at[b]).wait()
        pltpu.make_async_copy(rows_b.at[b], acc_b.at[didx(j)],
                              ssb.at[b]).wait()

        @pl.when(j + nb < NCHUNK)
        def _():
          pltpu.async_copy(ta_hbm.at[sidx(j + nb)], rows_a.at[b], gsa.at[b])
          pltpu.async_copy(tb_hbm.at[sidx(j + nb)], rows_b.at[b], gsb.at[b])

      return carry

    lax.fori_loop(0, NCHUNK // nb, body, 0)
    plsc.subcore_barrier()
    _writeout(acc_a, outa_hbm, c, s)
    _writeout(acc_b, outb_hbm, c, s)

  return spmm2


_sc_spmm_dual = _make_sc_spmm_dual(32, 16, 5)


_BN = 1000


def _mm1_body(x_ref, w_ref, o_ref):
  o_ref[...] = jnp.dot(x_ref[...], w_ref[...],
                       preferred_element_type=jnp.float32)


def _tc_mm1(x, w1):
  return pl.pallas_call(
      _mm1_body,
      grid=(N // _BN,),
      in_specs=[
          pl.BlockSpec((_BN, D), lambda i: (i, 0)),
          pl.BlockSpec((D, 32), lambda i: (0, 0)),
      ],
      out_specs=pl.BlockSpec((_BN, 32), lambda i: (i, 0)),
      out_shape=jax.ShapeDtypeStruct((N, 32), jnp.float32),
  )(x, w1)


def _stage_b_body(p0_ref, p1_ref, w2_ref, w3_ref, t2_ref, t3_ref):
  h = jnp.maximum(p0_ref[...] + p1_ref[...], 0.0)
  t2_ref[...] = jnp.dot(h, w2_ref[...], preferred_element_type=jnp.float32)
  t3_ref[...] = jnp.dot(h, w3_ref[...], preferred_element_type=jnp.float32)


def _tc_stage_b(p0, p1, w2, w3):
  return pl.pallas_call(
      _stage_b_body,
      grid=(N // _BN,),
      in_specs=[
          pl.BlockSpec((_BN, 32), lambda i: (i, 0)),
          pl.BlockSpec((_BN, 32), lambda i: (i, 0)),
          pl.BlockSpec((32, 32), lambda i: (0, 0)),
          pl.BlockSpec((32, 32), lambda i: (0, 0)),
      ],
      out_specs=[
          pl.BlockSpec((_BN, 32), lambda i: (i, 0)),
          pl.BlockSpec((_BN, 32), lambda i: (i, 0)),
      ],
      out_shape=[
          jax.ShapeDtypeStruct((N, 32), jnp.float32),
          jax.ShapeDtypeStruct((N, 32), jnp.float32),
      ],
  )(p0, p1, w2, w3)


def _stage_c_body(q0_ref, q1_ref, wd_ref, mu_ref, mub_ref, d_ref):
  mu = q0_ref[...] + q1_ref[...]
  mu_ref[...] = mu
  mub_ref[...] = mu.astype(jnp.bfloat16)
  d_ref[...] = jnp.dot(mu, wd_ref[...], preferred_element_type=jnp.float32)


def _tc_stage_c(q0, q1, wd16):
  return pl.pallas_call(
      _stage_c_body,
      grid=(N // _BN,),
      in_specs=[
          pl.BlockSpec((_BN, 32), lambda i: (i, 0)),
          pl.BlockSpec((_BN, 32), lambda i: (i, 0)),
          pl.BlockSpec((32, 16), lambda i: (0, 0)),
      ],
      out_specs=[
          pl.BlockSpec((_BN, 32), lambda i: (i, 0)),
          pl.BlockSpec((_BN, 32), lambda i: (i, 0)),
          pl.BlockSpec((_BN, 16), lambda i: (i, 0)),
      ],
      out_shape=[
          jax.ShapeDtypeStruct((N, 32), jnp.float32),
          jax.ShapeDtypeStruct((N, 32), jnp.bfloat16),
          jax.ShapeDtypeStruct((N, 16), jnp.float32),
      ],
  )(q0, q1, wd16)


_BDI = 512
_BDJ = 2048


def _dc_body(mu_i_ref, mu_j_ref, dc_ref):
  dc_ref[...] = lax.dot_general(
      mu_i_ref[...], mu_j_ref[...],
      dimension_numbers=(((1,), (1,)), ((), ())),
      preferred_element_type=jnp.float32)


def _tc_dc(mu_b):
  return pl.pallas_call(
      _dc_body,
      grid=(pl.cdiv(N, _BDI), pl.cdiv(N, _BDJ)),
      in_specs=[
          pl.BlockSpec((_BDI, 32), lambda i, j: (i, 0)),
          pl.BlockSpec((_BDJ, 32), lambda i, j: (j, 0)),
      ],
      out_specs=pl.BlockSpec((_BDI, _BDJ), lambda i, j: (i, j)),
      out_shape=jax.ShapeDtypeStruct((N, N), jnp.float32),
  )(mu_b, mu_b)


def _final_body(lv0_ref, lv1_ref, r0_ref, r1_ref, lv_ref, dec_ref):
  lv_ref[...] = lv0_ref[...] + lv1_ref[...]
  dec_ref[...] = (r0_ref[...] + r1_ref[...])[:, :1]


def _tc_final(lv0, lv1, r0, r1):
  return pl.pallas_call(
      _final_body,
      in_specs=[
          pl.BlockSpec((N, 32), lambda: (0, 0)),
          pl.BlockSpec((N, 32), lambda: (0, 0)),
          pl.BlockSpec((N, 16), lambda: (0, 0)),
          pl.BlockSpec((N, 16), lambda: (0, 0)),
      ],
      out_specs=[
          pl.BlockSpec((N, 32), lambda: (0, 0)),
          pl.BlockSpec((N, 1), lambda: (0, 0)),
      ],
      out_shape=[
          jax.ShapeDtypeStruct((N, 32), jnp.float32),
          jax.ShapeDtypeStruct((N, 1), jnp.float32),
      ],
  )(lv0, lv1, r0, r1)


def kernel(x, edge_index, W1, W2, W3, Wd):
  zeros32 = jnp.zeros((N, 32), jnp.float32)
  zeros16 = jnp.zeros((N, 16), jnp.float32)
  wd16 = jnp.tile(Wd, (1, 16))

  t1 = _tc_mm1(x, W1)
  p = _sc_spmm32(t1, edge_index, zeros32)
  t2, t3 = _tc_stage_b(p[0], p[1], W2, W3)
  q = _sc_spmm32(t2, edge_index, zeros32)
  mu, mu_b, d = _tc_stage_c(q[0], q[1], wd16)
  lv, r = _sc_spmm_dual(t3, d, edge_index, zeros32, zeros16)
  dc = _tc_dc(mu_b)
  logvar, dec = _tc_final(lv[0], lv[1], r[0], r[1])
  return (dec, dc, mu, logvar)


# speedup vs baseline: 16.5088x; 1.0150x over previous
import functools

import jax
import jax.numpy as jnp
from jax import lax
from jax.experimental import pallas as pl
from jax.experimental.pallas import tpu as pltpu
from jax.experimental.pallas import tpu_sc as plsc

N = 10000
E = 320000
D = 128

NC = 2
NS = 16
NW = NC * NS
EPW = E // NW
CHUNK = 80
NCHUNK = EPW // CHUNK
STRIPE = 624

_SC_PARAMS = pltpu.CompilerParams(use_tc_tiling_on_sc=False)


def _edge_prolog(edge_hbm, src_idx, dst_idx, wid):
  sl = pl.ds(wid * EPW, EPW)
  pltpu.sync_copy(edge_hbm.at[0, sl], src_idx)
  pltpu.sync_copy(edge_hbm.at[1, sl], dst_idx)


def _zero_acc(zeros_hbm, acc, s):
  stripe = pl.ds(s * STRIPE, STRIPE)
  tail = pl.ds(NS * STRIPE, N - NS * STRIPE)
  pltpu.sync_copy(zeros_hbm.at[stripe], acc.at[stripe])

  @pl.when(s == NS - 1)
  def _():
    pltpu.sync_copy(zeros_hbm.at[tail], acc.at[tail])


def _writeout(acc, out_hbm, c, s):
  stripe = pl.ds(s * STRIPE, STRIPE)
  tail = pl.ds(NS * STRIPE, N - NS * STRIPE)
  pltpu.sync_copy(acc.at[stripe], out_hbm.at[c, stripe])

  @pl.when(s == NS - 1)
  def _():
    pltpu.sync_copy(acc.at[tail], out_hbm.at[c, tail])


def _make_sc_spmm(width: int, nb: int):
  mesh = plsc.VectorSubcoreMesh(core_axis_name="c", subcore_axis_name="s")

  @functools.partial(
      pl.kernel,
      out_type=jax.ShapeDtypeStruct((NC, N, width), jnp.float32),
      mesh=mesh,
      scratch_types=[
          pltpu.VMEM((EPW,), jnp.int32),
          pltpu.VMEM((EPW,), jnp.int32),
          pltpu.VMEM((nb, CHUNK, width), jnp.float32),
          pltpu.VMEM_SHARED((N, width), jnp.float32),
          pltpu.SemaphoreType.DMA((nb,)),
          pltpu.SemaphoreType.DMA((nb,)),
      ],
      compiler_params=_SC_PARAMS,
  )
  def spmm(table_hbm, edge_hbm, zeros_hbm, out_hbm, src_idx, dst_idx,
           rows, acc, gsem, ssem):
    c = lax.axis_index("c")
    s = lax.axis_index("s")
    wid = s * NC + c
    _edge_prolog(edge_hbm, src_idx, dst_idx, wid)

    def sidx(j):
      return src_idx.at[pl.ds(j * CHUNK, CHUNK)]

    def didx(j):
      return dst_idx.at[pl.ds(j * CHUNK, CHUNK)]

    for b in range(nb):
      pltpu.async_copy(table_hbm.at[sidx(b)], rows.at[b], gsem.at[b])
    _zero_acc(zeros_hbm, acc, s)
    plsc.subcore_barrier()

    def body(g, carry):
      for b in range(nb):
        j = g * nb + b
        pltpu.make_async_copy(table_hbm.at[sidx(j)], rows.at[b],
                              gsem.at[b]).wait()
        pltpu.async_copy(rows.at[b], acc.at[didx(j)], ssem.at[b], add=True)
      for b in range(nb):
        j = g * nb + b
        pltpu.make_async_copy(rows.at[b], acc.at[didx(j)], ssem.at[b]).wait()

        @pl.when(j + nb < NCHUNK)
        def _():
          pltpu.async_copy(table_hbm.at[sidx(j + nb)], rows.at[b], gsem.at[b])

      return carry

    lax.fori_loop(0, NCHUNK // nb, body, 0)
    plsc.subcore_barrier()
    _writeout(acc, out_hbm, c, s)

  return spmm


_sc_spmm32 = _make_sc_spmm(32, 5)


def _make_sc_spmm_dual(wa: int, wb: int, nb: int):
  mesh = plsc.VectorSubcoreMesh(core_axis_name="c", subcore_axis_name="s")

  @functools.partial(
      pl.kernel,
      out_type=[
          jax.ShapeDtypeStruct((NC, N, wa), jnp.float32),
          jax.ShapeDtypeStruct((NC, N, wb), jnp.float32),
      ],
      mesh=mesh,
      scratch_types=[
          pltpu.VMEM((EPW,), jnp.int32),
          pltpu.VMEM((EPW,), jnp.int32),
          pltpu.VMEM((nb, CHUNK, wa), jnp.float32),
          pltpu.VMEM((nb, CHUNK, wb), jnp.float32),
          pltpu.VMEM_SHARED((N, wa), jnp.float32),
          pltpu.VMEM_SHARED((N, wb), jnp.float32),
          pltpu.SemaphoreType.DMA((nb,)),
          pltpu.SemaphoreType.DMA((nb,)),
          pltpu.SemaphoreType.DMA((nb,)),
          pltpu.SemaphoreType.DMA((nb,)),
      ],
      compiler_params=_SC_PARAMS,
  )
  def spmm2(ta_hbm, tb_hbm, edge_hbm, za_hbm, zb_hbm, outa_hbm, outb_hbm,
            src_idx, dst_idx, rows_a, rows_b, acc_a, acc_b,
            gsa, gsb, ssa, ssb):
    c = lax.axis_index("c")
    s = lax.axis_index("s")
    wid = s * NC + c
    _edge_prolog(edge_hbm, src_idx, dst_idx, wid)

    def sidx(j):
      return src_idx.at[pl.ds(j * CHUNK, CHUNK)]

    def didx(j):
      return dst_idx.at[pl.ds(j * CHUNK, CHUNK)]

    for b in range(nb):
      pltpu.async_copy(ta_hbm.at[sidx(b)], rows_a.at[b], gsa.at[b])
      pltpu.async_copy(tb_hbm.at[sidx(b)], rows_b.at[b], gsb.at[b])
    _zero_acc(za_hbm, acc_a, s)
    _zero_acc(zb_hbm, acc_b, s)
    plsc.subcore_barrier()

    def body(g, carry):
      for b in range(nb):
        j = g * nb + b
        pltpu.make_async_copy(ta_hbm.at[sidx(j)], rows_a.at[b],
                              gsa.at[b]).wait()
        pltpu.async_copy(rows_a.at[b], acc_a.at[didx(j)], ssa.at[b], add=True)
        pltpu.make_async_copy(tb_hbm.at[sidx(j)], rows_b.at[b],
                              gsb.at[b]).wait()
        pltpu.async_copy(rows_b.at[b], acc_b.at[didx(j)], ssb.at[b], add=True)
      for b in range(nb):
        j = g * nb + b
        pltpu.make_async_copy(rows_a.at[b], acc_a.at[didx(j)],
                              ssa.at[b]).wait()
        pltpu.make_async_copy(rows_b.at[b], acc_b.at[didx(j)],
                              ssb.at[b]).wait()

        @pl.when(j + nb < NCHUNK)
        def _():
          pltpu.async_copy(ta_hbm.at[sidx(j + nb)], rows_a.at[b], gsa.at[b])
          pltpu.async_copy(tb_hbm.at[sidx(j + nb)], rows_b.at[b], gsb.at[b])

      return carry

    lax.fori_loop(0, NCHUNK // nb, body, 0)
    plsc.subcore_barrier()
    _writeout(acc_a, outa_hbm, c, s)
    _writeout(acc_b, outb_hbm, c, s)

  return spmm2


_sc_spmm_dual = _make_sc_spmm_dual(32, 16, 5)


_BN = 1000


def _mm1_body(x_ref, w_ref, o_ref):
  o_ref[...] = jnp.dot(x_ref[...], w_ref[...],
                       preferred_element_type=jnp.float32)


def _tc_mm1(x, w1):
  return pl.pallas_call(
      _mm1_body,
      grid=(N // _BN,),
      in_specs=[
          pl.BlockSpec((_BN, D), lambda i: (i, 0)),
          pl.BlockSpec((D, 32), lambda i: (0, 0)),
      ],
      out_specs=pl.BlockSpec((_BN, 32), lambda i: (i, 0)),
      out_shape=jax.ShapeDtypeStruct((N, 32), jnp.float32),
  )(x, w1)


def _stage_b_body(p0_ref, p1_ref, w2_ref, w3_ref, t2_ref, t3_ref):
  h = jnp.maximum(p0_ref[...] + p1_ref[...], 0.0)
  t2_ref[...] = jnp.dot(h, w2_ref[...], preferred_element_type=jnp.float32)
  t3_ref[...] = jnp.dot(h, w3_ref[...], preferred_element_type=jnp.float32)


def _tc_stage_b(p0, p1, w2, w3):
  return pl.pallas_call(
      _stage_b_body,
      grid=(N // _BN,),
      in_specs=[
          pl.BlockSpec((_BN, 32), lambda i: (i, 0)),
          pl.BlockSpec((_BN, 32), lambda i: (i, 0)),
          pl.BlockSpec((32, 32), lambda i: (0, 0)),
          pl.BlockSpec((32, 32), lambda i: (0, 0)),
      ],
      out_specs=[
          pl.BlockSpec((_BN, 32), lambda i: (i, 0)),
          pl.BlockSpec((_BN, 32), lambda i: (i, 0)),
      ],
      out_shape=[
          jax.ShapeDtypeStruct((N, 32), jnp.float32),
          jax.ShapeDtypeStruct((N, 32), jnp.float32),
      ],
  )(p0, p1, w2, w3)


def _stage_c_body(q0_ref, q1_ref, wd_ref, mu_ref, mub_ref, d_ref):
  mu = q0_ref[...] + q1_ref[...]
  mu_ref[...] = mu
  mub_ref[...] = mu.astype(jnp.bfloat16)
  d_ref[...] = jnp.dot(mu, wd_ref[...], preferred_element_type=jnp.float32)


def _tc_stage_c(q0, q1, wd16):
  return pl.pallas_call(
      _stage_c_body,
      grid=(N // _BN,),
      in_specs=[
          pl.BlockSpec((_BN, 32), lambda i: (i, 0)),
          pl.BlockSpec((_BN, 32), lambda i: (i, 0)),
          pl.BlockSpec((32, 16), lambda i: (0, 0)),
      ],
      out_specs=[
          pl.BlockSpec((_BN, 32), lambda i: (i, 0)),
          pl.BlockSpec((_BN, 32), lambda i: (i, 0)),
          pl.BlockSpec((_BN, 16), lambda i: (i, 0)),
      ],
      out_shape=[
          jax.ShapeDtypeStruct((N, 32), jnp.float32),
          jax.ShapeDtypeStruct((N, 32), jnp.bfloat16),
          jax.ShapeDtypeStruct((N, 16), jnp.float32),
      ],
  )(q0, q1, wd16)


_BDI = 512
_BDJ = 4096


def _dc_body(mu_i_ref, mu_j_ref, dc_ref):
  dc_ref[...] = lax.dot_general(
      mu_i_ref[...], mu_j_ref[...],
      dimension_numbers=(((1,), (1,)), ((), ())),
      preferred_element_type=jnp.float32)


def _tc_dc(mu_b):
  return pl.pallas_call(
      _dc_body,
      grid=(pl.cdiv(N, _BDI), pl.cdiv(N, _BDJ)),
      in_specs=[
          pl.BlockSpec((_BDI, 32), lambda i, j: (i, 0)),
          pl.BlockSpec((_BDJ, 32), lambda i, j: (j, 0)),
      ],
      out_specs=pl.BlockSpec((_BDI, _BDJ), lambda i, j: (i, j)),
      out_shape=jax.ShapeDtypeStruct((N, N), jnp.float32),
  )(mu_b, mu_b)


def _final_body(lv0_ref, lv1_ref, r0_ref, r1_ref, lv_ref, dec_ref):
  lv_ref[...] = lv0_ref[...] + lv1_ref[...]
  dec_ref[...] = (r0_ref[...] + r1_ref[...])[:, :1]


def _tc_final(lv0, lv1, r0, r1):
  return pl.pallas_call(
      _final_body,
      in_specs=[
          pl.BlockSpec((N, 32), lambda: (0, 0)),
          pl.BlockSpec((N, 32), lambda: (0, 0)),
          pl.BlockSpec((N, 16), lambda: (0, 0)),
          pl.BlockSpec((N, 16), lambda: (0, 0)),
      ],
      out_specs=[
          pl.BlockSpec((N, 32), lambda: (0, 0)),
          pl.BlockSpec((N, 1), lambda: (0, 0)),
      ],
      out_shape=[
          jax.ShapeDtypeStruct((N, 32), jnp.float32),
          jax.ShapeDtypeStruct((N, 1), jnp.float32),
      ],
  )(lv0, lv1, r0, r1)


def kernel(x, edge_index, W1, W2, W3, Wd):
  zeros32 = jnp.zeros((N, 32), jnp.float32)
  zeros16 = jnp.zeros((N, 16), jnp.float32)
  wd16 = jnp.tile(Wd, (1, 16))

  t1 = _tc_mm1(x, W1)
  p = _sc_spmm32(t1, edge_index, zeros32)
  t2, t3 = _tc_stage_b(p[0], p[1], W2, W3)
  q = _sc_spmm32(t2, edge_index, zeros32)
  mu, mu_b, d = _tc_stage_c(q[0], q[1], wd16)
  lv, r = _sc_spmm_dual(t3, d, edge_index, zeros32, zeros16)
  dc = _tc_dc(mu_b)
  logvar, dec = _tc_final(lv[0], lv[1], r[0], r[1])
  return (dec, dc, mu, logvar)


# speedup vs baseline: 17.1292x; 1.0376x over previous
import functools

import jax
import jax.numpy as jnp
from jax import lax
from jax.experimental import pallas as pl
from jax.experimental.pallas import tpu as pltpu
from jax.experimental.pallas import tpu_sc as plsc

N = 10000
NP = 10240
PR = NP * 32 // 128
E = 320000
D = 128

NC = 2
NS = 16
NW = NC * NS
EPW = E // NW
CHUNK = 80
NCHUNK = EPW // CHUNK
STRIPE = 624

_SC_PARAMS = pltpu.CompilerParams(use_tc_tiling_on_sc=False)


def _edge_prolog(edge_hbm, src_idx, dst_idx, wid):
  sl = pl.ds(wid * EPW, EPW)
  pltpu.sync_copy(edge_hbm.at[0, sl], src_idx)
  pltpu.sync_copy(edge_hbm.at[1, sl], dst_idx)


def _zero_acc(zeros_hbm, acc, s):
  stripe = pl.ds(s * STRIPE, STRIPE)
  tail = pl.ds(NS * STRIPE, N - NS * STRIPE)
  pltpu.sync_copy(zeros_hbm.at[stripe], acc.at[stripe])

  @pl.when(s == NS - 1)
  def _():
    pltpu.sync_copy(zeros_hbm.at[tail], acc.at[tail])


def _writeout(acc, out_hbm, c, s):
  stripe = pl.ds(s * STRIPE, STRIPE)
  tail = pl.ds(NS * STRIPE, N - NS * STRIPE)
  pltpu.sync_copy(acc.at[stripe], out_hbm.at[c, stripe])

  @pl.when(s == NS - 1)
  def _():
    pltpu.sync_copy(acc.at[tail], out_hbm.at[c, tail])


def _make_sc_spmm(width: int, nb: int):
  mesh = plsc.VectorSubcoreMesh(core_axis_name="c", subcore_axis_name="s")

  @functools.partial(
      pl.kernel,
      out_type=jax.ShapeDtypeStruct((NC, NP, width), jnp.float32),
      mesh=mesh,
      scratch_types=[
          pltpu.VMEM((EPW,), jnp.int32),
          pltpu.VMEM((EPW,), jnp.int32),
          pltpu.VMEM((nb, CHUNK, width), jnp.float32),
          pltpu.VMEM_SHARED((N, width), jnp.float32),
          pltpu.SemaphoreType.DMA((nb,)),
          pltpu.SemaphoreType.DMA((nb,)),
      ],
      compiler_params=_SC_PARAMS,
  )
  def spmm(table_hbm, edge_hbm, zeros_hbm, out_hbm, src_idx, dst_idx,
           rows, acc, gsem, ssem):
    c = lax.axis_index("c")
    s = lax.axis_index("s")
    wid = s * NC + c
    _edge_prolog(edge_hbm, src_idx, dst_idx, wid)

    def sidx(j):
      return src_idx.at[pl.ds(j * CHUNK, CHUNK)]

    def didx(j):
      return dst_idx.at[pl.ds(j * CHUNK, CHUNK)]

    for b in range(nb):
      pltpu.async_copy(table_hbm.at[sidx(b)], rows.at[b], gsem.at[b])
    _zero_acc(zeros_hbm, acc, s)
    plsc.subcore_barrier()

    def body(g, carry):
      for b in range(nb):
        j = g * nb + b
        pltpu.make_async_copy(table_hbm.at[sidx(j)], rows.at[b],
                              gsem.at[b]).wait()
        pltpu.async_copy(rows.at[b], acc.at[didx(j)], ssem.at[b], add=True)
      for b in range(nb):
        j = g * nb + b
        pltpu.make_async_copy(rows.at[b], acc.at[didx(j)], ssem.at[b]).wait()

        @pl.when(j + nb < NCHUNK)
        def _():
          pltpu.async_copy(table_hbm.at[sidx(j + nb)], rows.at[b], gsem.at[b])

      return carry

    lax.fori_loop(0, NCHUNK // nb, body, 0)
    plsc.subcore_barrier()
    _writeout(acc, out_hbm, c, s)

  return spmm


_sc_spmm32 = _make_sc_spmm(32, 5)


def _make_sc_spmm_dual(wa: int, wb: int, nb: int):
  mesh = plsc.VectorSubcoreMesh(core_axis_name="c", subcore_axis_name="s")

  @functools.partial(
      pl.kernel,
      out_type=[
          jax.ShapeDtypeStruct((NC, NP, wa), jnp.float32),
          jax.ShapeDtypeStruct((NC, NP, wb), jnp.float32),
      ],
      mesh=mesh,
      scratch_types=[
          pltpu.VMEM((EPW,), jnp.int32),
          pltpu.VMEM((EPW,), jnp.int32),
          pltpu.VMEM((nb, CHUNK, wa), jnp.float32),
          pltpu.VMEM((nb, CHUNK, wb), jnp.float32),
          pltpu.VMEM_SHARED((N, wa), jnp.float32),
          pltpu.VMEM_SHARED((N, wb), jnp.float32),
          pltpu.SemaphoreType.DMA((nb,)),
          pltpu.SemaphoreType.DMA((nb,)),
          pltpu.SemaphoreType.DMA((nb,)),
          pltpu.SemaphoreType.DMA((nb,)),
      ],
      compiler_params=_SC_PARAMS,
  )
  def spmm2(ta_hbm, tb_hbm, edge_hbm, za_hbm, zb_hbm, outa_hbm, outb_hbm,
            src_idx, dst_idx, rows_a, rows_b, acc_a, acc_b,
            gsa, gsb, ssa, ssb):
    c = lax.axis_index("c")
    s = lax.axis_index("s")
    wid = s * NC + c
    _edge_prolog(edge_hbm, src_idx, dst_idx, wid)

    def sidx(j):
      return src_idx.at[pl.ds(j * CHUNK, CHUNK)]

    def didx(j):
      return dst_idx.at[pl.ds(j * CHUNK, CHUNK)]

    for b in range(nb):
      pltpu.async_copy(ta_hbm.at[sidx(b)], rows_a.at[b], gsa.at[b])
      pltpu.async_copy(tb_hbm.at[sidx(b)], rows_b.at[b], gsb.at[b])
    _zero_acc(za_hbm, acc_a, s)
    _zero_acc(zb_hbm, acc_b, s)
    plsc.subcore_barrier()

    def body(g, carry):
      for b in range(nb):
        j = g * nb + b
        pltpu.make_async_copy(ta_hbm.at[sidx(j)], rows_a.at[b],
                              gsa.at[b]).wait()
        pltpu.async_copy(rows_a.at[b], acc_a.at[didx(j)], ssa.at[b], add=True)
        pltpu.make_async_copy(tb_hbm.at[sidx(j)], rows_b.at[b],
                              gsb.at[b]).wait()
        pltpu.async_copy(rows_b.at[b], acc_b.at[didx(j)], ssb.at[b], add=True)
      for b in range(nb):
        j = g * nb + b
        pltpu.make_async_copy(rows_a.at[b], acc_a.at[didx(j)],
                              ssa.at[b]).wait()
        pltpu.make_async_copy(rows_b.at[b], acc_b.at[didx(j)],
                              ssb.at[b]).wait()

        @pl.when(j + nb < NCHUNK)
        def _():
          pltpu.async_copy(ta_hbm.at[sidx(j + nb)], rows_a.at[b], gsa.at[b])
          pltpu.async_copy(tb_hbm.at[sidx(j + nb)], rows_b.at[b], gsb.at[b])

      return carry

    lax.fori_loop(0, NCHUNK // nb, body, 0)
    plsc.subcore_barrier()
    _writeout(acc_a, outa_hbm, c, s)
    _writeout(acc_b, outb_hbm, c, s)

  return spmm2


_sc_spmm_dual = _make_sc_spmm_dual(32, 32, 5)


_BN = 1024
_BP = 256


def _mm1_body(x_ref, w_ref, o_ref):
  o_ref[...] = jnp.dot(x_ref[...], w_ref[...],
                       preferred_element_type=jnp.float32)


def _tc_mm1(x, w1):
  return pl.pallas_call(
      _mm1_body,
      grid=(NP // _BN,),
      in_specs=[
          pl.BlockSpec((_BN, D), lambda i: (i, 0)),
          pl.BlockSpec((D, 32), lambda i: (0, 0)),
      ],
      out_specs=pl.BlockSpec((_BN, 32), lambda i: (i, 0)),
      out_shape=jax.ShapeDtypeStruct((NP, 32), jnp.float32),
  )(x, w1)


def _stage_b_body(p0_ref, p1_ref, w2_ref, w3_ref, t2_ref, t3_ref):
  h = jnp.maximum(p0_ref[...] + p1_ref[...], 0.0)
  t2_ref[...] = jnp.dot(h, w2_ref[...], preferred_element_type=jnp.float32)
  t3_ref[...] = jnp.dot(h, w3_ref[...], preferred_element_type=jnp.float32)


def _tc_stage_b(p0, p1, w2bd, w3bd):
  return pl.pallas_call(
      _stage_b_body,
      grid=(PR // _BP,),
      in_specs=[
          pl.BlockSpec((_BP, 128), lambda i: (i, 0)),
          pl.BlockSpec((_BP, 128), lambda i: (i, 0)),
          pl.BlockSpec((128, 128), lambda i: (0, 0)),
          pl.BlockSpec((128, 128), lambda i: (0, 0)),
      ],
      out_specs=[
          pl.BlockSpec((_BP, 128), lambda i: (i, 0)),
          pl.BlockSpec((_BP, 128), lambda i: (i, 0)),
      ],
      out_shape=[
          jax.ShapeDtypeStruct((PR, 128), jnp.float32),
          jax.ShapeDtypeStruct((PR, 128), jnp.float32),
      ],
  )(p0, p1, w2bd, w3bd)


def _stage_c_body(q0_ref, q1_ref, wd_ref, mu_ref, mub_ref, d_ref):
  mu = q0_ref[...] + q1_ref[...]
  mu_ref[...] = mu
  mub_ref[...] = mu.astype(jnp.bfloat16)
  d_ref[...] = jnp.dot(mu, wd_ref[...], preferred_element_type=jnp.float32)


def _tc_stage_c(q0, q1, wdbd):
  return pl.pallas_call(
      _stage_c_body,
      grid=(PR // _BP,),
      in_specs=[
          pl.BlockSpec((_BP, 128), lambda i: (i, 0)),
          pl.BlockSpec((_BP, 128), lambda i: (i, 0)),
          pl.BlockSpec((128, 128), lambda i: (0, 0)),
      ],
      out_specs=[
          pl.BlockSpec((_BP, 128), lambda i: (i, 0)),
          pl.BlockSpec((_BP, 128), lambda i: (i, 0)),
          pl.BlockSpec((_BP, 128), lambda i: (i, 0)),
      ],
      out_shape=[
          jax.ShapeDtypeStruct((PR, 128), jnp.float32),
          jax.ShapeDtypeStruct((PR, 128), jnp.bfloat16),
          jax.ShapeDtypeStruct((PR, 128), jnp.float32),
      ],
  )(q0, q1, wdbd)


_BDI = 512
_BDJ = 4096


def _dc_body(mu_i_ref, mu_j_ref, dc_ref):
  dc_ref[...] = lax.dot_general(
      mu_i_ref[...], mu_j_ref[...],
      dimension_numbers=(((1,), (1,)), ((), ())),
      preferred_element_type=jnp.float32)


def _tc_dc(mu_b):
  return pl.pallas_call(
      _dc_body,
      grid=(pl.cdiv(N, _BDI), pl.cdiv(N, _BDJ)),
      in_specs=[
          pl.BlockSpec((_BDI, 32), lambda i, j: (i, 0)),
          pl.BlockSpec((_BDJ, 32), lambda i, j: (j, 0)),
      ],
      out_specs=pl.BlockSpec((_BDI, _BDJ), lambda i, j: (i, j)),
      out_shape=jax.ShapeDtypeStruct((N, N), jnp.float32),
  )(mu_b, mu_b)


def _final_body(lv0_ref, lv1_ref, r0_ref, r1_ref, lv_ref, dec_ref):
  lv_ref[...] = lv0_ref[...] + lv1_ref[...]
  dec_ref[...] = r0_ref[...] + r1_ref[...]


def _tc_final(lv0, lv1, r0, r1):
  return pl.pallas_call(
      _final_body,
      grid=(PR // _BP,),
      in_specs=[pl.BlockSpec((_BP, 128), lambda i: (i, 0))] * 4,
      out_specs=[
          pl.BlockSpec((_BP, 128), lambda i: (i, 0)),
          pl.BlockSpec((_BP, 128), lambda i: (i, 0)),
      ],
      out_shape=[
          jax.ShapeDtypeStruct((PR, 128), jnp.float32),
          jax.ShapeDtypeStruct((PR, 128), jnp.float32),
      ],
  )(lv0, lv1, r0, r1)


def kernel(x, edge_index, W1, W2, W3, Wd):
  f32 = jnp.float32
  zeros32 = jnp.zeros((N, 32), f32)
  eye4 = jnp.eye(4, dtype=f32)
  w2bd = jnp.kron(eye4, W2)
  w3bd = jnp.kron(eye4, W3)
  wdbd = jnp.kron(eye4, jnp.tile(Wd, (1, 32)))

  t1 = _tc_mm1(x, W1)
  p = _sc_spmm32(t1, edge_index, zeros32)
  pp = p.reshape(NC, PR, 128)
  t2p, t3p = _tc_stage_b(pp[0], pp[1], w2bd, w3bd)
  q = _sc_spmm32(t2p.reshape(NP, 32), edge_index, zeros32)
  qp = q.reshape(NC, PR, 128)
  mu_p, mub_p, d_p = _tc_stage_c(qp[0], qp[1], wdbd)
  lv, r = _sc_spmm_dual(t3p.reshape(NP, 32), d_p.reshape(NP, 32),
                        edge_index, zeros32, zeros32)
  mu_b = mub_p.reshape(NP, 32)[:N]
  dc = _tc_dc(mu_b)
  lvp = lv.reshape(NC, PR, 128)
  rp = r.reshape(NC, PR, 128)
  lv_s, dec_s = _tc_final(lvp[0], lvp[1], rp[0], rp[1])
  mu = mu_p.reshape(NP, 32)[:N]
  logvar = lv_s.reshape(NP, 32)[:N]
  dec = dec_s.reshape(NP, 32)[:N, :1]
  return (dec, dc, mu, logvar)


# speedup vs baseline: 19.7924x; 1.1555x over previous
import functools

import jax
import jax.numpy as jnp
from jax import lax
from jax.experimental import pallas as pl
from jax.experimental.pallas import tpu as pltpu
from jax.experimental.pallas import tpu_sc as plsc

N = 10000
NP = 10240
PR = NP * 32 // 128
E = 320000
D = 128

NC = 2
NS = 16
NW = NC * NS
EPW = E // NW
CHUNK = 80
NCHUNK = EPW // CHUNK
STRIPE = 624

_SC_PARAMS = pltpu.CompilerParams(use_tc_tiling_on_sc=False)


def _edge_prolog(edge_hbm, src_idx, dst_idx, wid):
  sl = pl.ds(wid * EPW, EPW)
  pltpu.sync_copy(edge_hbm.at[0, sl], src_idx)
  pltpu.sync_copy(edge_hbm.at[1, sl], dst_idx)


def _zero_acc(zeros_hbm, acc, s):
  stripe = pl.ds(s * STRIPE, STRIPE)
  tail = pl.ds(NS * STRIPE, N - NS * STRIPE)
  pltpu.sync_copy(zeros_hbm.at[stripe], acc.at[stripe])

  @pl.when(s == NS - 1)
  def _():
    pltpu.sync_copy(zeros_hbm.at[tail], acc.at[tail])


def _writeout(acc, out_hbm, c, s):
  stripe = pl.ds(s * STRIPE, STRIPE)
  tail = pl.ds(NS * STRIPE, N - NS * STRIPE)
  pltpu.sync_copy(acc.at[stripe], out_hbm.at[c, stripe])

  @pl.when(s == NS - 1)
  def _():
    pltpu.sync_copy(acc.at[tail], out_hbm.at[c, tail])


def _make_sc_spmm(width: int, nb: int):
  mesh = plsc.VectorSubcoreMesh(core_axis_name="c", subcore_axis_name="s")

  @functools.partial(
      pl.kernel,
      out_type=jax.ShapeDtypeStruct((NC, NP, width), jnp.float32),
      mesh=mesh,
      scratch_types=[
          pltpu.VMEM((EPW,), jnp.int32),
          pltpu.VMEM((EPW,), jnp.int32),
          pltpu.VMEM((nb, CHUNK, width), jnp.float32),
          pltpu.VMEM_SHARED((N, width), jnp.float32),
          pltpu.SemaphoreType.DMA((nb,)),
          pltpu.SemaphoreType.DMA((nb,)),
      ],
      compiler_params=_SC_PARAMS,
  )
  def spmm(table_hbm, edge_hbm, zeros_hbm, out_hbm, src_idx, dst_idx,
           rows, acc, gsem, ssem):
    c = lax.axis_index("c")
    s = lax.axis_index("s")
    wid = s * NC + c
    _edge_prolog(edge_hbm, src_idx, dst_idx, wid)

    def sidx(j):
      return src_idx.at[pl.ds(j * CHUNK, CHUNK)]

    def didx(j):
      return dst_idx.at[pl.ds(j * CHUNK, CHUNK)]

    for b in range(nb):
      pltpu.async_copy(table_hbm.at[sidx(b)], rows.at[b], gsem.at[b])
    _zero_acc(zeros_hbm, acc, s)
    plsc.subcore_barrier()

    def body(g, carry):
      for b in range(nb):
        j = g * nb + b
        pltpu.make_async_copy(table_hbm.at[sidx(j)], rows.at[b],
                              gsem.at[b]).wait()
        pltpu.async_copy(rows.at[b], acc.at[didx(j)], ssem.at[b], add=True)
      for b in range(nb):
        j = g * nb + b
        pltpu.make_async_copy(rows.at[b], acc.at[didx(j)], ssem.at[b]).wait()

        @pl.when(j + nb < NCHUNK)
        def _():
          pltpu.async_copy(table_hbm.at[sidx(j + nb)], rows.at[b], gsem.at[b])

      return carry

    lax.fori_loop(0, NCHUNK // nb, body, 0)
    plsc.subcore_barrier()
    _writeout(acc, out_hbm, c, s)

  return spmm


_sc_spmm32 = _make_sc_spmm(32, 5)


def _make_sc_spmm_dual(wa: int, wb: int, nb: int):
  mesh = plsc.VectorSubcoreMesh(core_axis_name="c", subcore_axis_name="s")

  @functools.partial(
      pl.kernel,
      out_type=[
          jax.ShapeDtypeStruct((NC, NP, wa), jnp.float32),
          jax.ShapeDtypeStruct((NC, NP, wb), jnp.float32),
      ],
      mesh=mesh,
      scratch_types=[
          pltpu.VMEM((EPW,), jnp.int32),
          pltpu.VMEM((EPW,), jnp.int32),
          pltpu.VMEM((nb, CHUNK, wa), jnp.float32),
          pltpu.VMEM((nb, CHUNK, wb), jnp.float32),
          pltpu.VMEM_SHARED((N, wa), jnp.float32),
          pltpu.VMEM_SHARED((N, wb), jnp.float32),
          pltpu.SemaphoreType.DMA((nb,)),
          pltpu.SemaphoreType.DMA((nb,)),
          pltpu.SemaphoreType.DMA((nb,)),
          pltpu.SemaphoreType.DMA((nb,)),
      ],
      compiler_params=_SC_PARAMS,
  )
  def spmm2(ta_hbm, tb_hbm, edge_hbm, za_hbm, zb_hbm, outa_hbm, outb_hbm,
            src_idx, dst_idx, rows_a, rows_b, acc_a, acc_b,
            gsa, gsb, ssa, ssb):
    c = lax.axis_index("c")
    s = lax.axis_index("s")
    wid = s * NC + c
    _edge_prolog(edge_hbm, src_idx, dst_idx, wid)

    def sidx(j):
      return src_idx.at[pl.ds(j * CHUNK, CHUNK)]

    def didx(j):
      return dst_idx.at[pl.ds(j * CHUNK, CHUNK)]

    for b in range(nb):
      pltpu.async_copy(ta_hbm.at[sidx(b)], rows_a.at[b], gsa.at[b])
      pltpu.async_copy(tb_hbm.at[sidx(b)], rows_b.at[b], gsb.at[b])
    _zero_acc(za_hbm, acc_a, s)
    _zero_acc(zb_hbm, acc_b, s)
    plsc.subcore_barrier()

    def body(g, carry):
      for b in range(nb):
        j = g * nb + b
        pltpu.make_async_copy(ta_hbm.at[sidx(j)], rows_a.at[b],
                              gsa.at[b]).wait()
        pltpu.async_copy(rows_a.at[b], acc_a.at[didx(j)], ssa.at[b], add=True)
        pltpu.make_async_copy(tb_hbm.at[sidx(j)], rows_b.at[b],
                              gsb.at[b]).wait()
        pltpu.async_copy(rows_b.at[b], acc_b.at[didx(j)], ssb.at[b], add=True)
      for b in range(nb):
        j = g * nb + b
        pltpu.make_async_copy(rows_a.at[b], acc_a.at[didx(j)],
                              ssa.at[b]).wait()
        pltpu.make_async_copy(rows_b.at[b], acc_b.at[didx(j)],
                              ssb.at[b]).wait()

        @pl.when(j + nb < NCHUNK)
        def _():
          pltpu.async_copy(ta_hbm.at[sidx(j + nb)], rows_a.at[b], gsa.at[b])
          pltpu.async_copy(tb_hbm.at[sidx(j + nb)], rows_b.at[b], gsb.at[b])

      return carry

    lax.fori_loop(0, NCHUNK // nb, body, 0)
    plsc.subcore_barrier()
    _writeout(acc_a, outa_hbm, c, s)
    _writeout(acc_b, outb_hbm, c, s)

  return spmm2


_sc_spmm_dual = _make_sc_spmm_dual(32, 32, 5)


_BN = 1024
_BP = 256


def _mm1_body(x_ref, w_ref, o_ref):
  o_ref[...] = jnp.dot(x_ref[...], w_ref[...],
                       preferred_element_type=jnp.float32)


def _tc_mm1(x, w1):
  return pl.pallas_call(
      _mm1_body,
      grid=(NP // _BN,),
      in_specs=[
          pl.BlockSpec((_BN, D), lambda i: (i, 0)),
          pl.BlockSpec((D, 32), lambda i: (0, 0)),
      ],
      out_specs=pl.BlockSpec((_BN, 32), lambda i: (i, 0)),
      out_shape=jax.ShapeDtypeStruct((NP, 32), jnp.float32),
  )(x, w1)


def _stage_b_body(p0_ref, p1_ref, w2_ref, w3_ref, t2_ref, t3_ref):
  h = jnp.maximum(p0_ref[0] + p1_ref[0], 0.0)
  t2_ref[...] = jnp.dot(h, w2_ref[...], preferred_element_type=jnp.float32)
  t3_ref[...] = jnp.dot(h, w3_ref[...], preferred_element_type=jnp.float32)


def _tc_stage_b(p, w2bd, w3bd):
  return pl.pallas_call(
      _stage_b_body,
      grid=(PR // _BP,),
      in_specs=[
          pl.BlockSpec((1, _BP, 128), lambda i: (0, i, 0)),
          pl.BlockSpec((1, _BP, 128), lambda i: (1, i, 0)),
          pl.BlockSpec((128, 128), lambda i: (0, 0)),
          pl.BlockSpec((128, 128), lambda i: (0, 0)),
      ],
      out_specs=[
          pl.BlockSpec((_BP, 128), lambda i: (i, 0)),
          pl.BlockSpec((_BP, 128), lambda i: (i, 0)),
      ],
      out_shape=[
          jax.ShapeDtypeStruct((PR, 128), jnp.float32),
          jax.ShapeDtypeStruct((PR, 128), jnp.float32),
      ],
  )(p, p, w2bd, w3bd)


def _stage_c_body(q0_ref, q1_ref, wd_ref, mu_ref, mub_ref, d_ref):
  mu = q0_ref[0] + q1_ref[0]
  mu_ref[...] = mu
  mub_ref[...] = mu.astype(jnp.bfloat16)
  d_ref[...] = jnp.dot(mu, wd_ref[...], preferred_element_type=jnp.float32)


def _tc_stage_c(q, wdbd):
  return pl.pallas_call(
      _stage_c_body,
      grid=(PR // _BP,),
      in_specs=[
          pl.BlockSpec((1, _BP, 128), lambda i: (0, i, 0)),
          pl.BlockSpec((1, _BP, 128), lambda i: (1, i, 0)),
          pl.BlockSpec((128, 128), lambda i: (0, 0)),
      ],
      out_specs=[
          pl.BlockSpec((_BP, 128), lambda i: (i, 0)),
          pl.BlockSpec((_BP, 128), lambda i: (i, 0)),
          pl.BlockSpec((_BP, 128), lambda i: (i, 0)),
      ],
      out_shape=[
          jax.ShapeDtypeStruct((PR, 128), jnp.float32),
          jax.ShapeDtypeStruct((PR, 128), jnp.bfloat16),
          jax.ShapeDtypeStruct((PR, 128), jnp.float32),
      ],
  )(q, q, wdbd)


_BDI = 512
_BDJ = 4096


def _dc_body(mu_i_ref, mu_j_ref, dc_ref):
  dc_ref[...] = lax.dot_general(
      mu_i_ref[...], mu_j_ref[...],
      dimension_numbers=(((1,), (1,)), ((), ())),
      preferred_element_type=jnp.float32)


def _tc_dc(mu_b):
  return pl.pallas_call(
      _dc_body,
      grid=(pl.cdiv(N, _BDI), pl.cdiv(N, _BDJ)),
      in_specs=[
          pl.BlockSpec((_BDI, 32), lambda i, j: (i, 0)),
          pl.BlockSpec((_BDJ, 32), lambda i, j: (j, 0)),
      ],
      out_specs=pl.BlockSpec((_BDI, _BDJ), lambda i, j: (i, j)),
      out_shape=jax.ShapeDtypeStruct((N, N), jnp.float32),
  )(mu_b, mu_b)


def _final_body(lv0_ref, lv1_ref, r0_ref, r1_ref, lv_ref, dec_ref):
  lv_ref[...] = lv0_ref[0] + lv1_ref[0]
  dec_ref[...] = r0_ref[0] + r1_ref[0]


def _tc_final(lv, r):
  return pl.pallas_call(
      _final_body,
      grid=(PR // _BP,),
      in_specs=[
          pl.BlockSpec((1, _BP, 128), lambda i: (0, i, 0)),
          pl.BlockSpec((1, _BP, 128), lambda i: (1, i, 0)),
          pl.BlockSpec((1, _BP, 128), lambda i: (0, i, 0)),
          pl.BlockSpec((1, _BP, 128), lambda i: (1, i, 0)),
      ],
      out_specs=[
          pl.BlockSpec((_BP, 128), lambda i: (i, 0)),
          pl.BlockSpec((_BP, 128), lambda i: (i, 0)),
      ],
      out_shape=[
          jax.ShapeDtypeStruct((PR, 128), jnp.float32),
          jax.ShapeDtypeStruct((PR, 128), jnp.float32),
      ],
  )(lv, lv, r, r)


def kernel(x, edge_index, W1, W2, W3, Wd):
  f32 = jnp.float32
  zeros32 = jnp.zeros((N, 32), f32)
  eye4 = jnp.eye(4, dtype=f32)
  w2bd = jnp.kron(eye4, W2)
  w3bd = jnp.kron(eye4, W3)
  wdbd = jnp.kron(eye4, jnp.tile(Wd, (1, 32)))

  t1 = _tc_mm1(x, W1)
  p = _sc_spmm32(t1, edge_index, zeros32)
  pp = p.reshape(NC, PR, 128)
  t2p, t3p = _tc_stage_b(pp, w2bd, w3bd)
  q = _sc_spmm32(t2p.reshape(NP, 32), edge_index, zeros32)
  qp = q.reshape(NC, PR, 128)
  mu_p, mub_p, d_p = _tc_stage_c(qp, wdbd)
  lv, r = _sc_spmm_dual(t3p.reshape(NP, 32), d_p.reshape(NP, 32),
                        edge_index, zeros32, zeros32)
  mu_b = mub_p.reshape(NP, 32)[:N]
  dc = _tc_dc(mu_b)
  lv_s, dec_s = _tc_final(lv.reshape(NC, PR, 128), r.reshape(NC, PR, 128))
  mu = mu_p.reshape(NP, 32)[:N]
  logvar = lv_s.reshape(NP, 32)[:N]
  dec = dec_s.reshape(NP, 32)[:N, :1]
  return (dec, dc, mu, logvar)
